# Initial kernel scaffold; baseline (speedup 1.0000x reference)
import jax
import jax.numpy as jnp
from jax.experimental import pallas as pl


def kernel(x, edge_index, edge_attr, u, enc_W, enc_b, gat0_lin_W, gat0_att_src, gat0_att_dst, gat0_att_edge, gat0_edge_W, gat0_bias, gat0_ln_g, gat0_ln_b, gat1_lin_W, gat1_att_src, gat1_att_dst, gat1_att_edge, gat1_edge_W, gat1_bias, gat1_ln_g, gat1_ln_b, gat2_lin_W, gat2_att_src, gat2_att_dst, gat2_att_edge, gat2_edge_W, gat2_bias, gat2_ln_g, gat2_ln_b, gp_W, gp_b, gp_ln_g, gp_ln_b, head_priority_W1, head_priority_b1, head_priority_W2, head_priority_b2, head_cooperation_W1, head_cooperation_b1, head_cooperation_W2, head_cooperation_b2, head_urgency_W1, head_urgency_b1, head_urgency_W2, head_urgency_b2, head_safety_W1, head_safety_b1, head_safety_W2, head_safety_b2, head_strategy_W1, head_strategy_b1, head_strategy_W2, head_strategy_b2, glob_W1, glob_b1, glob_W2, glob_b2):
    raise NotImplementedError("write your pallas kernel here")


# speedup vs baseline: 6.8538x; 6.8538x over previous
import functools

import jax
import jax.numpy as jnp
from jax import lax
from jax.experimental import pallas as pl
from jax.experimental.pallas import tpu as pltpu
from jax.experimental.pallas import tpu_sc as plsc

N0 = 50000
E0 = 800000
ND, ED, GD, HD = 15, 10, 8, 64
H, C, NL = 4, 16, 3
NB = 1024
EB = 4096
NP = 49 * NB
EP = 196 * EB
TRASH = NP - 1
NEG = -3.4e38
F32 = jnp.float32


def _enc_body(x_ref, w_ref, b_ref, o_ref):
    i = pl.program_id(0)
    y = jnp.dot(x_ref[...], w_ref[...], preferred_element_type=F32) + b_ref[...]
    y = jnp.maximum(y, 0.0)
    rid = i * NB + lax.broadcasted_iota(jnp.int32, (NB, 1), 0)
    o_ref[...] = jnp.where(rid < N0, y, 0.0)


def _enc(xp, enc_W, enc_b):
    return pl.pallas_call(
        _enc_body,
        grid=(NP // NB,),
        in_specs=[
            pl.BlockSpec((NB, ND), lambda i: (i, 0)),
            pl.BlockSpec((ND, HD), lambda i: (0, 0)),
            pl.BlockSpec((1, HD), lambda i: (0, 0)),
        ],
        out_specs=pl.BlockSpec((NB, HD), lambda i: (i, 0)),
        out_shape=jax.ShapeDtypeStruct((NP, HD), F32),
    )(xp, enc_W, enc_b.reshape(1, HD))


def _edge_body(ea_ref, w2_ref, eap_ref, a0_ref, a1_ref, a2_ref):
    i = pl.program_id(0)
    ea = ea_ref[...]
    tT = lax.dot_general(w2_ref[...], ea, (((0,), (1,)), ((), ())),
                         preferred_element_type=F32)
    rid = i * EB + lax.broadcasted_iota(jnp.int32, (EB, 1), 0)
    one = jnp.where(rid < E0, 1.0, 0.0)
    eap_ref[...] = jnp.concatenate(
        [ea, one, jnp.zeros((EB, 5), F32)], axis=1)
    a0_ref[...] = tT[0:4]
    a1_ref[...] = tT[4:8]
    a2_ref[...] = tT[8:12]


def _edge_prep(eap_raw, w2all):
    aspec = pl.BlockSpec((4, EB), lambda i: (0, i))
    return pl.pallas_call(
        _edge_body,
        grid=(EP // EB,),
        in_specs=[
            pl.BlockSpec((EB, ED), lambda i: (i, 0)),
            pl.BlockSpec((ED, 12), lambda i: (0, 0)),
        ],
        out_specs=[pl.BlockSpec((EB, 16), lambda i: (i, 0)), aspec, aspec, aspec],
        out_shape=[
            jax.ShapeDtypeStruct((EP, 16), F32),
            jax.ShapeDtypeStruct((4, EP), F32),
            jax.ShapeDtypeStruct((4, EP), F32),
            jax.ShapeDtypeStruct((4, EP), F32),
        ],
    )(eap_raw, w2all)


def _loopattr_body(t0_ref, t1_ref, o_ref):
    s = t0_ref[...] + t1_ref[...]
    deg = jnp.maximum(s[:, 10:11], 1.0)
    o_ref[...] = s / deg


def _loopattr(t):
    return pl.pallas_call(
        _loopattr_body,
        grid=(NP // NB,),
        in_specs=[
            pl.BlockSpec((NB, 16), lambda i: (i, 0)),
            pl.BlockSpec((NB, 16), lambda i: (i + NP // NB, 0)),
        ],
        out_specs=pl.BlockSpec((NB, 16), lambda i: (i, 0)),
        out_shape=jax.ShapeDtypeStruct((NP, 16), F32),
    )(t, t)


def _prep_body(x_ref, la_ref, w_ref, asf_ref, adf_ref, w2p_ref, s_ref,
               tab_ref, xhp_ref):
    xh = jnp.dot(x_ref[...], w_ref[...], preferred_element_type=F32)
    a_src = jnp.dot(xh * asf_ref[...], s_ref[...], preferred_element_type=F32)
    a_dst = jnp.dot(xh * adf_ref[...], s_ref[...], preferred_element_type=F32)
    ael = jnp.dot(la_ref[...], w2p_ref[...], preferred_element_type=F32)
    am = a_src + a_dst + ael
    m = jnp.where(am >= 0.0, am, 0.2 * am)
    tab_ref[...] = jnp.concatenate(
        [a_src, a_dst, m, jnp.zeros((NB, 4), F32)], axis=1)
    xhp_ref[0, :, :] = xh[:, 0:32]
    xhp_ref[1, :, :] = xh[:, 32:64]


def _prep(x, la, lin_W, asf, adf, w2pad, sind):
    return pl.pallas_call(
        _prep_body,
        grid=(NP // NB,),
        in_specs=[
            pl.BlockSpec((NB, HD), lambda i: (i, 0)),
            pl.BlockSpec((NB, 16), lambda i: (i, 0)),
            pl.BlockSpec((HD, HD), lambda i: (0, 0)),
            pl.BlockSpec((1, HD), lambda i: (0, 0)),
            pl.BlockSpec((1, HD), lambda i: (0, 0)),
            pl.BlockSpec((16, 4), lambda i: (0, 0)),
            pl.BlockSpec((HD, 4), lambda i: (0, 0)),
        ],
        out_specs=[
            pl.BlockSpec((NB, 16), lambda i: (i, 0)),
            pl.BlockSpec((2, NB, 32), lambda i: (0, i, 0)),
        ],
        out_shape=[
            jax.ShapeDtypeStruct((NP, 16), F32),
            jax.ShapeDtypeStruct((2, NP, 32), F32),
        ],
    )(x, la, lin_W, asf, adf, w2pad, sind)


def _combine_body(x_ref, xhp_ref, a0, a1, a2, a3, d0, d1, b_ref, g_ref,
                  bb_ref, o_ref):
    denom = d0[...] + d1[...] + 1.0
    accs = (a0, a1, a2, a3)
    msgs = []
    for h in range(4):
        xh_h = xhp_ref[h // 2, :, (h % 2) * 16:(h % 2) * 16 + 16]
        msgs.append((accs[h][...] + xh_h) / denom[:, h:h + 1])
    y = x_ref[...] + jnp.concatenate(msgs, axis=1) + b_ref[...]
    mean = jnp.mean(y, axis=1, keepdims=True)
    var = jnp.mean((y - mean) ** 2, axis=1, keepdims=True)
    o_ref[...] = (y - mean) * lax.rsqrt(var + 1e-5) * g_ref[...] + bb_ref[...]


def _combine(x, xhp, acc, dn, bias, ln_g, ln_b):
    nblk = NP // NB
    aspec = lambda h: pl.BlockSpec((NB, 16), lambda i, h=h: (i + h * nblk, 0))
    dspec = lambda c: pl.BlockSpec((NB, 4), lambda i, c=c: (i + c * nblk, 0))
    return pl.pallas_call(
        _combine_body,
        grid=(nblk,),
        in_specs=[
            pl.BlockSpec((NB, HD), lambda i: (i, 0)),
            pl.BlockSpec((2, NB, 32), lambda i: (0, i, 0)),
            aspec(0), aspec(1), aspec(2), aspec(3),
            dspec(0), dspec(1),
            pl.BlockSpec((1, HD), lambda i: (0, 0)),
            pl.BlockSpec((1, HD), lambda i: (0, 0)),
            pl.BlockSpec((1, HD), lambda i: (0, 0)),
        ],
        out_specs=pl.BlockSpec((NB, HD), lambda i: (i, 0)),
        out_shape=jax.ShapeDtypeStruct((NP, HD), F32),
    )(x, xhp, acc, acc, acc, acc, dn, dn,
      bias.reshape(1, HD), ln_g.reshape(1, HD), ln_b.reshape(1, HD))


def _sigmoid(z):
    return 1.0 / (1.0 + jnp.exp(-z))


def _final_body(x_ref, u_ref, gpw_ref, gpb_ref, gplg_ref, gplb_ref,
                w1_ref, b1_ref, w2p_ref, b2p_ref, w2c_ref, b2c_ref,
                w2u_ref, b2u_ref, w2f_ref, b2f_ref, w2s_ref, b2s_ref,
                gw1_ref, gb1_ref, gw2_ref, gb2_ref,
                pri_ref, coop_ref, urg_ref, saf_ref, strat_ref, gs_ref,
                sacc, xacc):
    i = pl.program_id(0)
    xb = x_ref[...]
    rid = i * NB + lax.broadcasted_iota(jnp.int32, (NB, 1), 0)
    valid = rid < N0

    @pl.when(i == 0)
    def _():
        sacc[...] = jnp.zeros_like(sacc)
        xacc[...] = jnp.full_like(xacc, NEG)

    sacc[0:1, :] += jnp.sum(jnp.where(valid, xb, 0.0), axis=0, keepdims=True)
    xacc[0:1, :] = jnp.maximum(
        xacc[0:1, :], jnp.max(jnp.where(valid, xb, NEG), axis=0, keepdims=True))

    hb = jnp.maximum(
        jnp.dot(xb, w1_ref[...], preferred_element_type=F32) + b1_ref[...], 0.0)
    pri_ref[...] = jnp.tanh(
        jnp.dot(hb[:, 0:32], w2p_ref[...], preferred_element_type=F32) + b2p_ref[...])
    coop_ref[...] = _sigmoid(
        jnp.dot(hb[:, 32:64], w2c_ref[...], preferred_element_type=F32) + b2c_ref[...])
    urg_ref[...] = _sigmoid(
        jnp.dot(hb[:, 64:96], w2u_ref[...], preferred_element_type=F32) + b2u_ref[...])
    saf_ref[...] = _sigmoid(
        jnp.dot(hb[:, 96:128], w2f_ref[...], preferred_element_type=F32) + b2f_ref[...])
    z = jnp.dot(hb[:, 128:160], w2s_ref[...], preferred_element_type=F32) + b2s_ref[...]
    zm = jnp.max(z, axis=1, keepdims=True)
    ez = jnp.exp(z - zm)
    strat_ref[...] = ez / jnp.sum(ez, axis=1, keepdims=True)

    @pl.when(i == NP // NB - 1)
    def _():
        ps = sacc[0:1, :]
        pm = ps / float(N0)
        px = xacc[0:1, :]
        gi = jnp.concatenate([pm, px, ps, u_ref[...]], axis=1)
        g0 = jnp.maximum(
            jnp.dot(gi, gpw_ref[...], preferred_element_type=F32) + gpb_ref[...], 0.0)
        mean = jnp.mean(g0, axis=1, keepdims=True)
        var = jnp.mean((g0 - mean) ** 2, axis=1, keepdims=True)
        g = (g0 - mean) * lax.rsqrt(var + 1e-5) * gplg_ref[...] + gplb_ref[...]
        gh = jnp.maximum(
            jnp.dot(g, gw1_ref[...], preferred_element_type=F32) + gb1_ref[...], 0.0)
        gs_ref[...] = jnp.tanh(
            jnp.dot(gh, gw2_ref[...], preferred_element_type=F32) + gb2_ref[...])


def _final(x, u, gp_W, gp_b, gp_ln_g, gp_ln_b, w1_all, b1_all, heads, glob):
    (w2p, b2p), (w2c, b2c), (w2u, b2u), (w2f, b2f), (w2s, b2s) = heads
    gw1, gb1, gw2, gb2 = glob
    full = lambda a, b: pl.BlockSpec((a, b), lambda i: (0, 0))
    return pl.pallas_call(
        _final_body,
        grid=(NP // NB,),
        in_specs=[
            pl.BlockSpec((NB, HD), lambda i: (i, 0)),
            full(1, GD), full(3 * HD + GD, GD), full(1, GD), full(1, GD),
            full(1, GD),
            full(HD, 160), full(1, 160),
            full(32, 1), full(1, 1), full(32, 1), full(1, 1),
            full(32, 1), full(1, 1), full(32, 1), full(1, 1),
            full(32, 5), full(1, 5),
            full(GD, 32), full(1, 32), full(32, 4), full(1, 4),
        ],
        out_specs=[
            pl.BlockSpec((NB, 1), lambda i: (i, 0)),
            pl.BlockSpec((NB, 1), lambda i: (i, 0)),
            pl.BlockSpec((NB, 1), lambda i: (i, 0)),
            pl.BlockSpec((NB, 1), lambda i: (i, 0)),
            pl.BlockSpec((NB, 5), lambda i: (i, 0)),
            pl.BlockSpec((1, 4), lambda i: (0, 0)),
        ],
        out_shape=[
            jax.ShapeDtypeStruct((NP, 1), F32),
            jax.ShapeDtypeStruct((NP, 1), F32),
            jax.ShapeDtypeStruct((NP, 1), F32),
            jax.ShapeDtypeStruct((NP, 1), F32),
            jax.ShapeDtypeStruct((NP, 5), F32),
            jax.ShapeDtypeStruct((1, 4), F32),
        ],
        scratch_shapes=[
            pltpu.VMEM((8, HD), F32),
            pltpu.VMEM((8, HD), F32),
        ],
    )(x, u.reshape(1, GD), gp_W, gp_b.reshape(1, GD),
      gp_ln_g.reshape(1, GD), gp_ln_b.reshape(1, GD), w1_all, b1_all,
      w2p, b2p.reshape(1, 1), w2c, b2c.reshape(1, 1),
      w2u, b2u.reshape(1, 1), w2f, b2f.reshape(1, 1),
      w2s, b2s.reshape(1, 5), gw1, gb1.reshape(1, 32), gw2, gb2.reshape(1, 4))


def _p0(eap, dpad, zeros16):
    t0 = jax.ops.segment_sum(eap, dpad, num_segments=NP)
    return jnp.concatenate([t0, jnp.zeros((NP, 16), F32)], axis=0)


def _pa(spad, dpad, table, aeT, zeros16):
    a_src = table[spad, 0:4]
    a_dst = table[dpad, 4:8]
    m = table[dpad, 8:12]
    al = a_src + a_dst + aeT.T
    al = jnp.where(al >= 0.0, al, 0.2 * al)
    ev = jnp.exp(al - m)
    dn = jax.ops.segment_sum(ev, dpad, num_segments=NP)
    return ev.T, jnp.concatenate([dn, jnp.zeros((NP, 4), F32)], axis=0)


def _pb(spad, dpad, xhp, ev, zeros16):
    accs = []
    for h in range(4):
        xh_h = xhp[h // 2, :, (h % 2) * 16:(h % 2) * 16 + 16]
        msg = xh_h[spad] * ev[h][:, None]
        accs.append(jax.ops.segment_sum(msg, dpad, num_segments=NP))
    return jnp.concatenate(accs, axis=0)


def kernel(x, edge_index, edge_attr, u, enc_W, enc_b,
           gat0_lin_W, gat0_att_src, gat0_att_dst, gat0_att_edge, gat0_edge_W,
           gat0_bias, gat0_ln_g, gat0_ln_b,
           gat1_lin_W, gat1_att_src, gat1_att_dst, gat1_att_edge, gat1_edge_W,
           gat1_bias, gat1_ln_g, gat1_ln_b,
           gat2_lin_W, gat2_att_src, gat2_att_dst, gat2_att_edge, gat2_edge_W,
           gat2_bias, gat2_ln_g, gat2_ln_b,
           gp_W, gp_b, gp_ln_g, gp_ln_b,
           head_priority_W1, head_priority_b1, head_priority_W2, head_priority_b2,
           head_cooperation_W1, head_cooperation_b1, head_cooperation_W2, head_cooperation_b2,
           head_urgency_W1, head_urgency_b1, head_urgency_W2, head_urgency_b2,
           head_safety_W1, head_safety_b1, head_safety_W2, head_safety_b2,
           head_strategy_W1, head_strategy_b1, head_strategy_W2, head_strategy_b2,
           glob_W1, glob_b1, glob_W2, glob_b2):
    gat = [
        (gat0_lin_W, gat0_att_src, gat0_att_dst, gat0_att_edge, gat0_edge_W,
         gat0_bias, gat0_ln_g, gat0_ln_b),
        (gat1_lin_W, gat1_att_src, gat1_att_dst, gat1_att_edge, gat1_edge_W,
         gat1_bias, gat1_ln_g, gat1_ln_b),
        (gat2_lin_W, gat2_att_src, gat2_att_dst, gat2_att_edge, gat2_edge_W,
         gat2_bias, gat2_ln_g, gat2_ln_b),
    ]

    xp = jnp.pad(x, ((0, NP - N0), (0, 0)))
    spad = jnp.concatenate(
        [edge_index[0], jnp.zeros((EP - E0,), jnp.int32)])
    dpad = jnp.concatenate(
        [edge_index[1], jnp.full((EP - E0,), TRASH, jnp.int32)])
    eap_raw = jnp.pad(edge_attr, ((0, EP - E0), (0, 0)))
    zeros16 = jnp.zeros((NP, 16), F32)

    w2s_l = [(gw[4].reshape(ED, H, C) * gw[3][None]).sum(-1) for gw in gat]
    w2all = jnp.concatenate(w2s_l, axis=1)
    w2pad = [jnp.pad(w2, ((0, 6), (0, 0))) for w2 in w2s_l]
    sind = jnp.repeat(jnp.eye(4, dtype=F32), 16, axis=0)

    eap, aeT0, aeT1, aeT2 = _edge_prep(eap_raw, w2all)
    aeTs = [aeT0, aeT1, aeT2]
    t = _p0(eap, dpad, zeros16)
    la = _loopattr(t)

    xcur = _enc(xp, enc_W, enc_b)
    for l in range(NL):
        lin_W, att_src, att_dst, att_edge, edge_W, bias, ln_g, ln_b = gat[l]
        tab, xhp = _prep(xcur, la, lin_W,
                         att_src.reshape(1, HD), att_dst.reshape(1, HD),
                         w2pad[l], sind)
        ev, dn = _pa(spad, dpad, tab, aeTs[l], zeros16)
        acc = _pb(spad, dpad, xhp, ev, zeros16)
        xcur = _combine(xcur, xhp, acc, dn, bias, ln_g, ln_b)

    w1_all = jnp.concatenate(
        [head_priority_W1, head_cooperation_W1, head_urgency_W1,
         head_safety_W1, head_strategy_W1], axis=1)
    b1_all = jnp.concatenate(
        [head_priority_b1, head_cooperation_b1, head_urgency_b1,
         head_safety_b1, head_strategy_b1]).reshape(1, 160)
    heads = [(head_priority_W2, head_priority_b2),
             (head_cooperation_W2, head_cooperation_b2),
             (head_urgency_W2, head_urgency_b2),
             (head_safety_W2, head_safety_b2),
             (head_strategy_W2, head_strategy_b2)]
    glob = (glob_W1, glob_b1, glob_W2, glob_b2)
    pri, coop, urg, saf, strat, gs = _final(
        xcur, u, gp_W, gp_b, gp_ln_g, gp_ln_b, w1_all, b1_all, heads, glob)
    return (pri[:N0], coop[:N0], urg[:N0], saf[:N0], strat[:N0],
            gs.reshape(GD // 2))


# speedup vs baseline: 36.8348x; 5.3744x over previous
import dataclasses
import functools

import jax
import jax.numpy as jnp
from jax import lax
from jax.experimental import pallas as pl
from jax.experimental.pallas import tpu as pltpu
from jax.experimental.pallas import tpu_sc as plsc

N0 = 50000
E0 = 800000
ND, ED, GD, HD = 15, 10, 8, 64
H, C, NL = 4, 16, 3
NB = 1024
EB = 4096
NP = 49 * NB
EP = 196 * EB
TRASH = NP - 1
NEG = -3.4e38
F32 = jnp.float32


def _enc_body(x_ref, w_ref, b_ref, o_ref):
    i = pl.program_id(0)
    y = jnp.dot(x_ref[...], w_ref[...], preferred_element_type=F32) + b_ref[...]
    y = jnp.maximum(y, 0.0)
    rid = i * NB + lax.broadcasted_iota(jnp.int32, (NB, 1), 0)
    o_ref[...] = jnp.where(rid < N0, y, 0.0)


def _enc(xp, enc_W, enc_b):
    return pl.pallas_call(
        _enc_body,
        grid=(NP // NB,),
        in_specs=[
            pl.BlockSpec((NB, ND), lambda i: (i, 0)),
            pl.BlockSpec((ND, HD), lambda i: (0, 0)),
            pl.BlockSpec((1, HD), lambda i: (0, 0)),
        ],
        out_specs=pl.BlockSpec((NB, HD), lambda i: (i, 0)),
        out_shape=jax.ShapeDtypeStruct((NP, HD), F32),
    )(xp, enc_W, enc_b.reshape(1, HD))


def _edge_body(ea_ref, w2_ref, eap_ref, a0_ref, a1_ref, a2_ref):
    i = pl.program_id(0)
    ea = ea_ref[...]
    tT = lax.dot_general(w2_ref[...], ea, (((0,), (1,)), ((), ())),
                         preferred_element_type=F32)
    rid = i * EB + lax.broadcasted_iota(jnp.int32, (EB, 1), 0)
    one = jnp.where(rid < E0, 1.0, 0.0)
    eap_ref[...] = jnp.concatenate(
        [ea, one, jnp.zeros((EB, 5), F32)], axis=1)
    a0_ref[...] = tT[0:4]
    a1_ref[...] = tT[4:8]
    a2_ref[...] = tT[8:12]


def _edge_prep(eap_raw, w2all):
    aspec = pl.BlockSpec((4, EB), lambda i: (0, i))
    return pl.pallas_call(
        _edge_body,
        grid=(EP // EB,),
        in_specs=[
            pl.BlockSpec((EB, ED), lambda i: (i, 0)),
            pl.BlockSpec((ED, 12), lambda i: (0, 0)),
        ],
        out_specs=[pl.BlockSpec((EB, 16), lambda i: (i, 0)), aspec, aspec, aspec],
        out_shape=[
            jax.ShapeDtypeStruct((EP, 16), F32),
            jax.ShapeDtypeStruct((4, EP), F32),
            jax.ShapeDtypeStruct((4, EP), F32),
            jax.ShapeDtypeStruct((4, EP), F32),
        ],
    )(eap_raw, w2all)


def _loopattr_body(t0_ref, t1_ref, o_ref):
    s = t0_ref[...] + t1_ref[...]
    deg = jnp.maximum(s[:, 10:11], 1.0)
    o_ref[...] = s / deg


def _loopattr(t):
    return pl.pallas_call(
        _loopattr_body,
        grid=(NP // NB,),
        in_specs=[
            pl.BlockSpec((NB, 16), lambda i: (i, 0)),
            pl.BlockSpec((NB, 16), lambda i: (i + NP // NB, 0)),
        ],
        out_specs=pl.BlockSpec((NB, 16), lambda i: (i, 0)),
        out_shape=jax.ShapeDtypeStruct((NP, 16), F32),
    )(t, t)


def _prep_body(x_ref, la_ref, w_ref, asf_ref, adf_ref, w2p_ref, s_ref,
               tab_ref, xhp_ref):
    xh = jnp.dot(x_ref[...], w_ref[...], preferred_element_type=F32)
    a_src = jnp.dot(xh * asf_ref[...], s_ref[...], preferred_element_type=F32)
    a_dst = jnp.dot(xh * adf_ref[...], s_ref[...], preferred_element_type=F32)
    ael = jnp.dot(la_ref[...], w2p_ref[...], preferred_element_type=F32)
    am = a_src + a_dst + ael
    m = jnp.where(am >= 0.0, am, 0.2 * am)
    tab_ref[...] = jnp.concatenate(
        [a_src, a_dst, m, jnp.zeros((NB, 4), F32)], axis=1)
    xhp_ref[0, :, :] = xh[:, 0:32]
    xhp_ref[1, :, :] = xh[:, 32:64]


def _prep(x, la, lin_W, asf, adf, w2pad, sind):
    return pl.pallas_call(
        _prep_body,
        grid=(NP // NB,),
        in_specs=[
            pl.BlockSpec((NB, HD), lambda i: (i, 0)),
            pl.BlockSpec((NB, 16), lambda i: (i, 0)),
            pl.BlockSpec((HD, HD), lambda i: (0, 0)),
            pl.BlockSpec((1, HD), lambda i: (0, 0)),
            pl.BlockSpec((1, HD), lambda i: (0, 0)),
            pl.BlockSpec((16, 4), lambda i: (0, 0)),
            pl.BlockSpec((HD, 4), lambda i: (0, 0)),
        ],
        out_specs=[
            pl.BlockSpec((NB, 16), lambda i: (i, 0)),
            pl.BlockSpec((2, NB, 32), lambda i: (0, i, 0)),
        ],
        out_shape=[
            jax.ShapeDtypeStruct((NP, 16), F32),
            jax.ShapeDtypeStruct((2, NP, 32), F32),
        ],
    )(x, la, lin_W, asf, adf, w2pad, sind)


def _combine_body(x_ref, xhp_ref, a0, a1, a2, a3, d0, d1, b_ref, g_ref,
                  bb_ref, o_ref):
    denom = d0[...][:, 0:4] + d1[...][:, 0:4] + 1.0
    accs = (a0, a1, a2, a3)
    msgs = []
    for h in range(4):
        xh_h = xhp_ref[h // 2, :, (h % 2) * 16:(h % 2) * 16 + 16]
        msgs.append((accs[h][...] + xh_h) / denom[:, h:h + 1])
    y = x_ref[...] + jnp.concatenate(msgs, axis=1) + b_ref[...]
    mean = jnp.mean(y, axis=1, keepdims=True)
    var = jnp.mean((y - mean) ** 2, axis=1, keepdims=True)
    o_ref[...] = (y - mean) * lax.rsqrt(var + 1e-5) * g_ref[...] + bb_ref[...]


def _combine(x, xhp, acc, dn, bias, ln_g, ln_b):
    nblk = NP // NB
    aspec = lambda h: pl.BlockSpec((NB, 16), lambda i, h=h: (i + h * nblk, 0))
    dspec = lambda c: pl.BlockSpec((NB, 16), lambda i, c=c: (i + c * nblk, 0))
    return pl.pallas_call(
        _combine_body,
        grid=(nblk,),
        in_specs=[
            pl.BlockSpec((NB, HD), lambda i: (i, 0)),
            pl.BlockSpec((2, NB, 32), lambda i: (0, i, 0)),
            aspec(0), aspec(1), aspec(2), aspec(3),
            dspec(0), dspec(1),
            pl.BlockSpec((1, HD), lambda i: (0, 0)),
            pl.BlockSpec((1, HD), lambda i: (0, 0)),
            pl.BlockSpec((1, HD), lambda i: (0, 0)),
        ],
        out_specs=pl.BlockSpec((NB, HD), lambda i: (i, 0)),
        out_shape=jax.ShapeDtypeStruct((NP, HD), F32),
    )(x, xhp, acc, acc, acc, acc, dn, dn,
      bias.reshape(1, HD), ln_g.reshape(1, HD), ln_b.reshape(1, HD))


def _sigmoid(z):
    return 1.0 / (1.0 + jnp.exp(-z))


def _final_body(x_ref, u_ref, gpw_ref, gpb_ref, gplg_ref, gplb_ref,
                w1_ref, b1_ref, w2p_ref, b2p_ref, w2c_ref, b2c_ref,
                w2u_ref, b2u_ref, w2f_ref, b2f_ref, w2s_ref, b2s_ref,
                gw1_ref, gb1_ref, gw2_ref, gb2_ref,
                pri_ref, coop_ref, urg_ref, saf_ref, strat_ref, gs_ref,
                sacc, xacc):
    i = pl.program_id(0)
    xb = x_ref[...]
    rid = i * NB + lax.broadcasted_iota(jnp.int32, (NB, 1), 0)
    valid = rid < N0

    @pl.when(i == 0)
    def _():
        sacc[...] = jnp.zeros_like(sacc)
        xacc[...] = jnp.full_like(xacc, NEG)

    sacc[0:1, :] += jnp.sum(jnp.where(valid, xb, 0.0), axis=0, keepdims=True)
    xacc[0:1, :] = jnp.maximum(
        xacc[0:1, :], jnp.max(jnp.where(valid, xb, NEG), axis=0, keepdims=True))

    hb = jnp.maximum(
        jnp.dot(xb, w1_ref[...], preferred_element_type=F32) + b1_ref[...], 0.0)
    pri_ref[...] = jnp.tanh(
        jnp.dot(hb[:, 0:32], w2p_ref[...], preferred_element_type=F32) + b2p_ref[...])
    coop_ref[...] = _sigmoid(
        jnp.dot(hb[:, 32:64], w2c_ref[...], preferred_element_type=F32) + b2c_ref[...])
    urg_ref[...] = _sigmoid(
        jnp.dot(hb[:, 64:96], w2u_ref[...], preferred_element_type=F32) + b2u_ref[...])
    saf_ref[...] = _sigmoid(
        jnp.dot(hb[:, 96:128], w2f_ref[...], preferred_element_type=F32) + b2f_ref[...])
    z = jnp.dot(hb[:, 128:160], w2s_ref[...], preferred_element_type=F32) + b2s_ref[...]
    zm = jnp.max(z, axis=1, keepdims=True)
    ez = jnp.exp(z - zm)
    strat_ref[...] = ez / jnp.sum(ez, axis=1, keepdims=True)

    @pl.when(i == NP // NB - 1)
    def _():
        ps = sacc[0:1, :]
        pm = ps / float(N0)
        px = xacc[0:1, :]
        gi = jnp.concatenate([pm, px, ps, u_ref[...]], axis=1)
        g0 = jnp.maximum(
            jnp.dot(gi, gpw_ref[...], preferred_element_type=F32) + gpb_ref[...], 0.0)
        mean = jnp.mean(g0, axis=1, keepdims=True)
        var = jnp.mean((g0 - mean) ** 2, axis=1, keepdims=True)
        g = (g0 - mean) * lax.rsqrt(var + 1e-5) * gplg_ref[...] + gplb_ref[...]
        gh = jnp.maximum(
            jnp.dot(g, gw1_ref[...], preferred_element_type=F32) + gb1_ref[...], 0.0)
        gs_ref[...] = jnp.tanh(
            jnp.dot(gh, gw2_ref[...], preferred_element_type=F32) + gb2_ref[...])


def _final(x, u, gp_W, gp_b, gp_ln_g, gp_ln_b, w1_all, b1_all, heads, glob):
    (w2p, b2p), (w2c, b2c), (w2u, b2u), (w2f, b2f), (w2s, b2s) = heads
    gw1, gb1, gw2, gb2 = glob
    full = lambda a, b: pl.BlockSpec((a, b), lambda i: (0, 0))
    return pl.pallas_call(
        _final_body,
        grid=(NP // NB,),
        in_specs=[
            pl.BlockSpec((NB, HD), lambda i: (i, 0)),
            full(1, GD), full(3 * HD + GD, GD), full(1, GD), full(1, GD),
            full(1, GD),
            full(HD, 160), full(1, 160),
            full(32, 1), full(1, 1), full(32, 1), full(1, 1),
            full(32, 1), full(1, 1), full(32, 1), full(1, 1),
            full(32, 5), full(1, 5),
            full(GD, 32), full(1, 32), full(32, 4), full(1, 4),
        ],
        out_specs=[
            pl.BlockSpec((NB, 1), lambda i: (i, 0)),
            pl.BlockSpec((NB, 1), lambda i: (i, 0)),
            pl.BlockSpec((NB, 1), lambda i: (i, 0)),
            pl.BlockSpec((NB, 1), lambda i: (i, 0)),
            pl.BlockSpec((NB, 5), lambda i: (i, 0)),
            pl.BlockSpec((1, 4), lambda i: (0, 0)),
        ],
        out_shape=[
            jax.ShapeDtypeStruct((NP, 1), F32),
            jax.ShapeDtypeStruct((NP, 1), F32),
            jax.ShapeDtypeStruct((NP, 1), F32),
            jax.ShapeDtypeStruct((NP, 1), F32),
            jax.ShapeDtypeStruct((NP, 5), F32),
            jax.ShapeDtypeStruct((1, 4), F32),
        ],
        scratch_shapes=[
            pltpu.VMEM((8, HD), F32),
            pltpu.VMEM((8, HD), F32),
        ],
    )(x, u.reshape(1, GD), gp_W, gp_b.reshape(1, GD),
      gp_ln_g.reshape(1, GD), gp_ln_b.reshape(1, GD), w1_all, b1_all,
      w2p, b2p.reshape(1, 1), w2c, b2c.reshape(1, 1),
      w2u, b2u.reshape(1, 1), w2f, b2f.reshape(1, 1),
      w2s, b2s.reshape(1, 5), gw1, gb1.reshape(1, 32), gw2, gb2.reshape(1, 4))


_MESH = plsc.VectorSubcoreMesh(core_axis_name="c", subcore_axis_name="s")
_CP = pltpu.CompilerParams(use_tc_tiling_on_sc=False)
if "needs_layout_passes" in pltpu.CompilerParams.__dataclass_fields__:
    _CP = dataclasses.replace(_CP, needs_layout_passes=False)
KC = 128
RPS = NP // 16
CH0 = EP // (32 * KC)
CHB = EP // (16 * KC)


@functools.partial(
    pl.kernel,
    out_type=jax.ShapeDtypeStruct((2 * NP, 16), F32),
    mesh=_MESH,
    compiler_params=_CP,
    scratch_types=[
        pltpu.VMEM((KC,), jnp.int32),
        pltpu.VMEM((KC, 16), F32),
        pltpu.VMEM_SHARED((NP, 16), F32),
    ],
)
def _p0_sc(eap_hbm, dpad_hbm, zeros_hbm, out_hbm, idx_v, row_v, table_sh):
    c = lax.axis_index("c")
    s = lax.axis_index("s")
    w = c * 16 + s
    r0 = s * RPS
    pltpu.sync_copy(zeros_hbm.at[pl.ds(r0, RPS)], table_sh.at[pl.ds(r0, RPS)])
    plsc.subcore_barrier()

    @pl.loop(0, CH0)
    def _(t):
        base = (w * CH0 + t) * KC
        pltpu.sync_copy(dpad_hbm.at[pl.ds(base, KC)], idx_v)
        pltpu.sync_copy(eap_hbm.at[pl.ds(base, KC)], row_v)
        pltpu.sync_copy(row_v, table_sh.at[idx_v], add=True)

    plsc.subcore_barrier()
    pltpu.sync_copy(table_sh.at[pl.ds(r0, RPS)],
                    out_hbm.at[pl.ds(c * NP + r0, RPS)])


def _p0_jnp(eap, dpad, zeros16):
    t0 = jax.ops.segment_sum(eap, dpad, num_segments=NP)
    return jnp.concatenate([t0, jnp.zeros((NP, 16), F32)], axis=0)


def _pa_jnp(spad, dpad, table, aeT, zeros4):
    a_src = table[spad, 0:4]
    a_dst = table[dpad, 4:8]
    m = table[dpad, 8:12]
    al = a_src + a_dst + aeT.T
    al = jnp.where(al >= 0.0, al, 0.2 * al)
    ev = jnp.exp(al - m)
    dn = jax.ops.segment_sum(ev, dpad, num_segments=NP)
    dn = jnp.pad(dn, ((0, 0), (0, 12)))
    return ev.T, jnp.concatenate([dn, jnp.zeros((NP, 16), F32)], axis=0)


def _pb_jnp(spad, dpad, xhp, ev, zeros16):
    accs = []
    for h in range(4):
        xh_h = xhp[h // 2, :, (h % 2) * 16:(h % 2) * 16 + 16]
        msg = xh_h[spad] * ev[h][:, None]
        accs.append(jax.ops.segment_sum(msg, dpad, num_segments=NP))
    return jnp.concatenate(accs, axis=0)


def _p0(eap, dpad, zeros16):
    return _p0_sc(eap, dpad, zeros16)


@functools.partial(
    pl.kernel,
    out_type=[
        jax.ShapeDtypeStruct((4, EP), F32),
        jax.ShapeDtypeStruct((2 * NP, 16), F32),
    ],
    mesh=_MESH,
    compiler_params=_CP,
    scratch_types=[
        pltpu.VMEM((KC,), jnp.int32), pltpu.VMEM((KC,), jnp.int32),
        pltpu.VMEM((KC,), jnp.int32), pltpu.VMEM((KC,), jnp.int32),
        pltpu.VMEM((KC, 16), F32), pltpu.VMEM((KC, 16), F32),
        pltpu.VMEM((KC, 16), F32), pltpu.VMEM((KC, 16), F32),
        pltpu.VMEM((4, KC), F32), pltpu.VMEM((4, KC), F32),
        pltpu.VMEM((4, KC), F32), pltpu.VMEM((4, KC), F32),
        pltpu.VMEM((KC, 16), F32),
        pltpu.VMEM_SHARED((NP, 16), F32),
        pltpu.SemaphoreType.DMA, pltpu.SemaphoreType.DMA,
        pltpu.SemaphoreType.DMA, pltpu.SemaphoreType.DMA,
        pltpu.SemaphoreType.DMA, pltpu.SemaphoreType.DMA,
        pltpu.SemaphoreType.DMA, pltpu.SemaphoreType.DMA,
        pltpu.SemaphoreType.DMA, pltpu.SemaphoreType.DMA,
        pltpu.SemaphoreType.DMA, pltpu.SemaphoreType.DMA,
    ],
)
def _pa_sc(spad_hbm, dpad_hbm, table_hbm, aeT_hbm, zeros16_hbm,
           ev_hbm, dn_hbm,
           sidx0, sidx1, didx0, didx1, srow0, srow1, drow0, drow1,
           aebuf0, aebuf1, evbuf0, evbuf1, dnbuf, dnsh,
           sS0, sS1, sD0, sD1, sA0, sA1, sGS0, sGS1, sGD0, sGD1, sE0, sE1):
    c = lax.axis_index("c")
    s = lax.axis_index("s")
    w = c * 16 + s
    r0 = s * RPS
    iota = lax.iota(jnp.int32, 16)
    sidx = (sidx0, sidx1)
    didx = (didx0, didx1)
    srow = (srow0, srow1)
    drow = (drow0, drow1)
    aebuf = (aebuf0, aebuf1)
    evbuf = (evbuf0, evbuf1)
    sS, sD, sA = (sS0, sS1), (sD0, sD1), (sA0, sA1)
    sGS, sGD, sE = (sGS0, sGS1), (sGD0, sGD1), (sE0, sE1)
    pltpu.sync_copy(zeros16_hbm.at[pl.ds(r0, RPS)], dnsh.at[pl.ds(r0, RPS)])
    pltpu.sync_copy(zeros16_hbm.at[pl.ds(0, KC)], dnbuf)
    plsc.subcore_barrier()

    def s_desc(t, b):
        base = (w * CH0 + t) * KC
        return pltpu.make_async_copy(
            spad_hbm.at[pl.ds(base, KC)], sidx[b], sS[b])

    def d_desc(t, b):
        base = (w * CH0 + t) * KC
        return pltpu.make_async_copy(
            dpad_hbm.at[pl.ds(base, KC)], didx[b], sD[b])

    def a_desc(t, b):
        base = (w * CH0 + t) * KC
        return pltpu.make_async_copy(
            aeT_hbm.at[:, pl.ds(base, KC)], aebuf[b], sA[b])

    def gs_desc(b):
        return pltpu.make_async_copy(table_hbm.at[sidx[b]], srow[b], sGS[b])

    def gd_desc(b):
        return pltpu.make_async_copy(table_hbm.at[didx[b]], drow[b], sGD[b])

    def e_desc(t, b):
        base = (w * CH0 + t) * KC
        return pltpu.make_async_copy(
            evbuf[b], ev_hbm.at[:, pl.ds(base, KC)], sE[b])

    def issue_inputs(t, b):
        s_desc(t, b).start()
        d_desc(t, b).start()
        a_desc(t, b).start()

    def start_gathers(b):
        gs_desc(b).start()
        gd_desc(b).start()

    issue_inputs(0, 0)
    issue_inputs(1, 1)
    s_desc(0, 0).wait()
    d_desc(0, 0).wait()
    start_gathers(0)

    @pl.loop(0, CH0 // 2)
    def _(tt):
        for b in range(2):
            t = tt * 2 + b

            @pl.when(t + 1 < CH0)
            def _():
                s_desc(t + 1, 1 - b).wait()
                d_desc(t + 1, 1 - b).wait()
                start_gathers(1 - b)

            gs_desc(b).wait()
            gd_desc(b).wait()
            a_desc(t, b).wait()

            @pl.when(t >= 2)
            def _():
                e_desc(t - 2, b).wait()

            for g in range(8):
                rows = g * 16 + iota
                for h in range(4):
                    hcol = jnp.full((16,), h, jnp.int32)
                    asrc = plsc.load_gather(srow[b], [rows, hcol])
                    adst = plsc.load_gather(drow[b], [rows, hcol + 4])
                    mh = plsc.load_gather(drow[b], [rows, hcol + 8])
                    ae = aebuf[b][h, pl.ds(g * 16, 16)]
                    al = asrc + adst + ae
                    al = jnp.where(al >= 0.0, al, 0.2 * al)
                    evh = jnp.exp(al - mh)
                    evbuf[b][h, pl.ds(g * 16, 16)] = evh
                    plsc.store_scatter(dnbuf, [rows, hcol], evh)
            e_desc(t, b).start()
            pltpu.sync_copy(dnbuf, dnsh.at[didx[b]], add=True)

            @pl.when(t + 2 < CH0)
            def _():
                issue_inputs(t + 2, b)

    e_desc(CH0 - 2, 0).wait()
    e_desc(CH0 - 1, 1).wait()
    plsc.subcore_barrier()
    pltpu.sync_copy(dnsh.at[pl.ds(r0, RPS)],
                    dn_hbm.at[pl.ds(c * NP + r0, RPS)])


def _pa(spad, dpad, table, aeT, zeros16):
    return _pa_sc(spad, dpad, table, aeT, zeros16)


@functools.partial(
    pl.kernel,
    out_type=jax.ShapeDtypeStruct((4 * NP, 16), F32),
    mesh=_MESH,
    compiler_params=_CP,
    scratch_types=[
        pltpu.VMEM((KC,), jnp.int32), pltpu.VMEM((KC,), jnp.int32),
        pltpu.VMEM((KC,), jnp.int32), pltpu.VMEM((KC,), jnp.int32),
        pltpu.VMEM((KC, 32), F32), pltpu.VMEM((KC, 32), F32),
        pltpu.VMEM((KC,), F32), pltpu.VMEM((KC,), F32),
        pltpu.VMEM((KC,), F32), pltpu.VMEM((KC,), F32),
        pltpu.VMEM((KC, 16), F32),
        pltpu.VMEM((KC, 16), F32),
        pltpu.VMEM_SHARED((NP, 16), F32),
        pltpu.VMEM_SHARED((NP, 16), F32),
        pltpu.SemaphoreType.DMA, pltpu.SemaphoreType.DMA,
        pltpu.SemaphoreType.DMA, pltpu.SemaphoreType.DMA,
        pltpu.SemaphoreType.DMA, pltpu.SemaphoreType.DMA,
        pltpu.SemaphoreType.DMA, pltpu.SemaphoreType.DMA,
        pltpu.SemaphoreType.DMA, pltpu.SemaphoreType.DMA,
        pltpu.SemaphoreType.DMA, pltpu.SemaphoreType.DMA,
    ],
)
def _pb_sc(spad2_hbm, dpad_hbm, xhpf_hbm, ev1_hbm, zeros_hbm,
           acc_hbm,
           sidx0, sidx1, didx0, didx1, xrow0, xrow1,
           ev00, ev01, ev10, ev11, msg0, msg1, acc0_sh, acc1_sh,
           sS0, sS1, sD0, sD1, sX0, sX1, sE00, sE01, sE10, sE11, sM0, sM1):
    c = lax.axis_index("c")
    s = lax.axis_index("s")
    r0 = s * RPS
    iota = lax.iota(jnp.int32, 16)
    sidx = (sidx0, sidx1)
    didx = (didx0, didx1)
    xrow = (xrow0, xrow1)
    ev0 = (ev00, ev01)
    ev1 = (ev10, ev11)
    sS, sD, sX = (sS0, sS1), (sD0, sD1), (sX0, sX1)
    sE0, sE1 = (sE00, sE01), (sE10, sE11)
    pltpu.sync_copy(zeros_hbm.at[pl.ds(r0, RPS)], acc0_sh.at[pl.ds(r0, RPS)])
    pltpu.sync_copy(zeros_hbm.at[pl.ds(r0, RPS)], acc1_sh.at[pl.ds(r0, RPS)])
    plsc.subcore_barrier()

    def s_desc(t, b):
        base = (s * CHB + t) * KC
        return pltpu.make_async_copy(
            spad2_hbm.at[pl.ds(c * EP + base, KC)], sidx[b], sS[b])

    def d_desc(t, b):
        base = (s * CHB + t) * KC
        return pltpu.make_async_copy(
            dpad_hbm.at[pl.ds(base, KC)], didx[b], sD[b])

    def e_descs(t, b):
        base = (s * CHB + t) * KC
        return (pltpu.make_async_copy(
                    ev1_hbm.at[pl.ds((2 * c) * EP + base, KC)], ev0[b], sE0[b]),
                pltpu.make_async_copy(
                    ev1_hbm.at[pl.ds((2 * c + 1) * EP + base, KC)], ev1[b], sE1[b]))

    def x_desc(b):
        return pltpu.make_async_copy(xhpf_hbm.at[sidx[b]], xrow[b], sX[b])

    def issue_inputs(t, b):
        s_desc(t, b).start()
        d_desc(t, b).start()
        ea, eb = e_descs(t, b)
        ea.start()
        eb.start()

    issue_inputs(0, 0)
    issue_inputs(1, 1)
    s_desc(0, 0).wait()
    x_desc(0).start()

    @pl.loop(0, CHB // 2)
    def _(tt):
        for b in range(2):
            t = tt * 2 + b

            @pl.when(t + 1 < CHB)
            def _():
                s_desc(t + 1, 1 - b).wait()
                x_desc(1 - b).start()

            x_desc(b).wait()
            ea, eb = e_descs(t, b)
            ea.wait()
            eb.wait()
            for g in range(8):
                rows = g * 16 + iota
                ev0v = ev0[b][pl.ds(g * 16, 16)]
                ev1v = ev1[b][pl.ds(g * 16, 16)]
                for cc in range(16):
                    ccol = jnp.full((16,), cc, jnp.int32)
                    c0 = plsc.load_gather(xrow[b], [rows, ccol])
                    plsc.store_scatter(msg0, [rows, ccol], c0 * ev0v)
                    c1 = plsc.load_gather(xrow[b], [rows, ccol + 16])
                    plsc.store_scatter(msg1, [rows, ccol], c1 * ev1v)
            d_desc(t, b).wait()
            m0 = pltpu.make_async_copy(msg0, acc0_sh.at[didx[b]], sM0)
            m1 = pltpu.make_async_copy(msg1, acc1_sh.at[didx[b]], sM1)
            m0.start(add=True)
            m1.start(add=True)
            m0.wait()
            m1.wait()

            @pl.when(t + 2 < CHB)
            def _():
                issue_inputs(t + 2, b)

    plsc.subcore_barrier()
    pltpu.sync_copy(acc0_sh.at[pl.ds(r0, RPS)],
                    acc_hbm.at[pl.ds((2 * c) * NP + r0, RPS)])
    pltpu.sync_copy(acc1_sh.at[pl.ds(r0, RPS)],
                    acc_hbm.at[pl.ds((2 * c + 1) * NP + r0, RPS)])


def _pb(spad2, dpad, xhp, ev, zeros16):
    return _pb_sc(spad2, dpad, xhp.reshape(2 * NP, 32), ev.reshape(4 * EP),
                  zeros16)


def kernel(x, edge_index, edge_attr, u, enc_W, enc_b,
           gat0_lin_W, gat0_att_src, gat0_att_dst, gat0_att_edge, gat0_edge_W,
           gat0_bias, gat0_ln_g, gat0_ln_b,
           gat1_lin_W, gat1_att_src, gat1_att_dst, gat1_att_edge, gat1_edge_W,
           gat1_bias, gat1_ln_g, gat1_ln_b,
           gat2_lin_W, gat2_att_src, gat2_att_dst, gat2_att_edge, gat2_edge_W,
           gat2_bias, gat2_ln_g, gat2_ln_b,
           gp_W, gp_b, gp_ln_g, gp_ln_b,
           head_priority_W1, head_priority_b1, head_priority_W2, head_priority_b2,
           head_cooperation_W1, head_cooperation_b1, head_cooperation_W2, head_cooperation_b2,
           head_urgency_W1, head_urgency_b1, head_urgency_W2, head_urgency_b2,
           head_safety_W1, head_safety_b1, head_safety_W2, head_safety_b2,
           head_strategy_W1, head_strategy_b1, head_strategy_W2, head_strategy_b2,
           glob_W1, glob_b1, glob_W2, glob_b2):
    gat = [
        (gat0_lin_W, gat0_att_src, gat0_att_dst, gat0_att_edge, gat0_edge_W,
         gat0_bias, gat0_ln_g, gat0_ln_b),
        (gat1_lin_W, gat1_att_src, gat1_att_dst, gat1_att_edge, gat1_edge_W,
         gat1_bias, gat1_ln_g, gat1_ln_b),
        (gat2_lin_W, gat2_att_src, gat2_att_dst, gat2_att_edge, gat2_edge_W,
         gat2_bias, gat2_ln_g, gat2_ln_b),
    ]

    xp = jnp.pad(x, ((0, NP - N0), (0, 0)))
    spad = jnp.concatenate(
        [edge_index[0], jnp.zeros((EP - E0,), jnp.int32)])
    dpad = jnp.concatenate(
        [edge_index[1], jnp.full((EP - E0,), TRASH, jnp.int32)])
    spad2 = jnp.concatenate([spad, spad + NP])
    eap_raw = jnp.pad(edge_attr, ((0, EP - E0), (0, 0)))
    zeros16 = jnp.zeros((NP, 16), F32)

    w2s_l = [(gw[4].reshape(ED, H, C) * gw[3][None]).sum(-1) for gw in gat]
    w2all = jnp.concatenate(w2s_l, axis=1)
    w2pad = [jnp.pad(w2, ((0, 6), (0, 0))) for w2 in w2s_l]
    sind = jnp.repeat(jnp.eye(4, dtype=F32), 16, axis=0)

    eap, aeT0, aeT1, aeT2 = _edge_prep(eap_raw, w2all)
    aeTs = [aeT0, aeT1, aeT2]
    t = _p0(eap, dpad, zeros16)
    la = _loopattr(t)

    xcur = _enc(xp, enc_W, enc_b)
    for l in range(NL):
        lin_W, att_src, att_dst, att_edge, edge_W, bias, ln_g, ln_b = gat[l]
        tab, xhp = _prep(xcur, la, lin_W,
                         att_src.reshape(1, HD), att_dst.reshape(1, HD),
                         w2pad[l], sind)
        ev, dn = _pa(spad, dpad, tab, aeTs[l], zeros16)
        acc = _pb(spad2, dpad, xhp, ev, zeros16)
        xcur = _combine(xcur, xhp, acc, dn, bias, ln_g, ln_b)

    w1_all = jnp.concatenate(
        [head_priority_W1, head_cooperation_W1, head_urgency_W1,
         head_safety_W1, head_strategy_W1], axis=1)
    b1_all = jnp.concatenate(
        [head_priority_b1, head_cooperation_b1, head_urgency_b1,
         head_safety_b1, head_strategy_b1]).reshape(1, 160)
    heads = [(head_priority_W2, head_priority_b2),
             (head_cooperation_W2, head_cooperation_b2),
             (head_urgency_W2, head_urgency_b2),
             (head_safety_W2, head_safety_b2),
             (head_strategy_W2, head_strategy_b2)]
    glob = (glob_W1, glob_b1, glob_W2, glob_b2)
    pri, coop, urg, saf, strat, gs = _final(
        xcur, u, gp_W, gp_b, gp_ln_g, gp_ln_b, w1_all, b1_all, heads, glob)
    return (pri[:N0], coop[:N0], urg[:N0], saf[:N0], strat[:N0],
            gs.reshape(GD // 2))


# speedup vs baseline: 37.8328x; 1.0271x over previous
import dataclasses
import functools

import jax
import jax.numpy as jnp
from jax import lax
from jax.experimental import pallas as pl
from jax.experimental.pallas import tpu as pltpu
from jax.experimental.pallas import tpu_sc as plsc

N0 = 50000
E0 = 800000
ND, ED, GD, HD = 15, 10, 8, 64
H, C, NL = 4, 16, 3
NB = 1024
EB = 4096
NP = 49 * NB
EP = 196 * EB
TRASH = NP - 1
NEG = -3.4e38
F32 = jnp.float32


def _enc_body(x_ref, w_ref, b_ref, o_ref):
    i = pl.program_id(0)
    y = jnp.dot(x_ref[...], w_ref[...], preferred_element_type=F32) + b_ref[...]
    y = jnp.maximum(y, 0.0)
    rid = i * NB + lax.broadcasted_iota(jnp.int32, (NB, 1), 0)
    o_ref[...] = jnp.where(rid < N0, y, 0.0)


def _enc(xp, enc_W, enc_b):
    return pl.pallas_call(
        _enc_body,
        grid=(NP // NB,),
        in_specs=[
            pl.BlockSpec((NB, ND), lambda i: (i, 0)),
            pl.BlockSpec((ND, HD), lambda i: (0, 0)),
            pl.BlockSpec((1, HD), lambda i: (0, 0)),
        ],
        out_specs=pl.BlockSpec((NB, HD), lambda i: (i, 0)),
        out_shape=jax.ShapeDtypeStruct((NP, HD), F32),
    )(xp, enc_W, enc_b.reshape(1, HD))


def _edge_body(ea_ref, w2_ref, eap_ref, a0_ref, a1_ref, a2_ref):
    i = pl.program_id(0)
    ea = ea_ref[...]
    tT = lax.dot_general(w2_ref[...], ea, (((0,), (1,)), ((), ())),
                         preferred_element_type=F32)
    rid = i * EB + lax.broadcasted_iota(jnp.int32, (EB, 1), 0)
    one = jnp.where(rid < E0, 1.0, 0.0)
    eap_ref[...] = jnp.concatenate(
        [ea, one, jnp.zeros((EB, 5), F32)], axis=1)
    a0_ref[...] = tT[0:4]
    a1_ref[...] = tT[4:8]
    a2_ref[...] = tT[8:12]


def _edge_prep(eap_raw, w2all):
    aspec = pl.BlockSpec((4, EB), lambda i: (0, i))
    return pl.pallas_call(
        _edge_body,
        grid=(EP // EB,),
        in_specs=[
            pl.BlockSpec((EB, ED), lambda i: (i, 0)),
            pl.BlockSpec((ED, 12), lambda i: (0, 0)),
        ],
        out_specs=[pl.BlockSpec((EB, 16), lambda i: (i, 0)), aspec, aspec, aspec],
        out_shape=[
            jax.ShapeDtypeStruct((EP, 16), F32),
            jax.ShapeDtypeStruct((4, EP), F32),
            jax.ShapeDtypeStruct((4, EP), F32),
            jax.ShapeDtypeStruct((4, EP), F32),
        ],
    )(eap_raw, w2all)


def _loopattr_body(t0_ref, t1_ref, o_ref):
    s = t0_ref[...] + t1_ref[...]
    deg = jnp.maximum(s[:, 10:11], 1.0)
    o_ref[...] = s / deg


def _loopattr(t):
    return pl.pallas_call(
        _loopattr_body,
        grid=(NP // NB,),
        in_specs=[
            pl.BlockSpec((NB, 16), lambda i: (i, 0)),
            pl.BlockSpec((NB, 16), lambda i: (i + NP // NB, 0)),
        ],
        out_specs=pl.BlockSpec((NB, 16), lambda i: (i, 0)),
        out_shape=jax.ShapeDtypeStruct((NP, 16), F32),
    )(t, t)


def _prep_body(x_ref, la_ref, w_ref, asf_ref, adf_ref, w2p_ref, s_ref,
               tab_ref, xhp_ref):
    xh = jnp.dot(x_ref[...], w_ref[...], preferred_element_type=F32)
    a_src = jnp.dot(xh * asf_ref[...], s_ref[...], preferred_element_type=F32)
    a_dst = jnp.dot(xh * adf_ref[...], s_ref[...], preferred_element_type=F32)
    ael = jnp.dot(la_ref[...], w2p_ref[...], preferred_element_type=F32)
    am = a_src + a_dst + ael
    m = jnp.where(am >= 0.0, am, 0.2 * am)
    tab_ref[...] = jnp.concatenate(
        [a_src, a_dst, m, jnp.zeros((NB, 4), F32)], axis=1)
    xhp_ref[0, :, :] = xh[:, 0:32]
    xhp_ref[1, :, :] = xh[:, 32:64]


def _prep(x, la, lin_W, asf, adf, w2pad, sind):
    return pl.pallas_call(
        _prep_body,
        grid=(NP // NB,),
        in_specs=[
            pl.BlockSpec((NB, HD), lambda i: (i, 0)),
            pl.BlockSpec((NB, 16), lambda i: (i, 0)),
            pl.BlockSpec((HD, HD), lambda i: (0, 0)),
            pl.BlockSpec((1, HD), lambda i: (0, 0)),
            pl.BlockSpec((1, HD), lambda i: (0, 0)),
            pl.BlockSpec((16, 4), lambda i: (0, 0)),
            pl.BlockSpec((HD, 4), lambda i: (0, 0)),
        ],
        out_specs=[
            pl.BlockSpec((NB, 16), lambda i: (i, 0)),
            pl.BlockSpec((2, NB, 32), lambda i: (0, i, 0)),
        ],
        out_shape=[
            jax.ShapeDtypeStruct((NP, 16), F32),
            jax.ShapeDtypeStruct((2, NP, 32), F32),
        ],
    )(x, la, lin_W, asf, adf, w2pad, sind)


def _combine_body(x_ref, xhp_ref, a0, a1, a2, a3, d0, d1, b_ref, g_ref,
                  bb_ref, o_ref):
    denom = d0[...][:, 0:4] + d1[...][:, 0:4] + 1.0
    accs = (a0, a1, a2, a3)
    msgs = []
    for h in range(4):
        xh_h = xhp_ref[h // 2, :, (h % 2) * 16:(h % 2) * 16 + 16]
        msgs.append((accs[h][...] + xh_h) / denom[:, h:h + 1])
    y = x_ref[...] + jnp.concatenate(msgs, axis=1) + b_ref[...]
    mean = jnp.mean(y, axis=1, keepdims=True)
    var = jnp.mean((y - mean) ** 2, axis=1, keepdims=True)
    o_ref[...] = (y - mean) * lax.rsqrt(var + 1e-5) * g_ref[...] + bb_ref[...]


def _combine(x, xhp, acc, dn, bias, ln_g, ln_b):
    nblk = NP // NB
    aspec = lambda h: pl.BlockSpec((NB, 16), lambda i, h=h: (i + h * nblk, 0))
    dspec = lambda c: pl.BlockSpec((NB, 16), lambda i, c=c: (i + c * nblk, 0))
    return pl.pallas_call(
        _combine_body,
        grid=(nblk,),
        in_specs=[
            pl.BlockSpec((NB, HD), lambda i: (i, 0)),
            pl.BlockSpec((2, NB, 32), lambda i: (0, i, 0)),
            aspec(0), aspec(1), aspec(2), aspec(3),
            dspec(0), dspec(1),
            pl.BlockSpec((1, HD), lambda i: (0, 0)),
            pl.BlockSpec((1, HD), lambda i: (0, 0)),
            pl.BlockSpec((1, HD), lambda i: (0, 0)),
        ],
        out_specs=pl.BlockSpec((NB, HD), lambda i: (i, 0)),
        out_shape=jax.ShapeDtypeStruct((NP, HD), F32),
    )(x, xhp, acc, acc, acc, acc, dn, dn,
      bias.reshape(1, HD), ln_g.reshape(1, HD), ln_b.reshape(1, HD))


def _sigmoid(z):
    return 1.0 / (1.0 + jnp.exp(-z))


def _final_body(x_ref, u_ref, gpw_ref, gpb_ref, gplg_ref, gplb_ref,
                w1_ref, b1_ref, w2p_ref, b2p_ref, w2c_ref, b2c_ref,
                w2u_ref, b2u_ref, w2f_ref, b2f_ref, w2s_ref, b2s_ref,
                gw1_ref, gb1_ref, gw2_ref, gb2_ref,
                pri_ref, coop_ref, urg_ref, saf_ref, strat_ref, gs_ref,
                sacc, xacc):
    i = pl.program_id(0)
    xb = x_ref[...]
    rid = i * NB + lax.broadcasted_iota(jnp.int32, (NB, 1), 0)
    valid = rid < N0

    @pl.when(i == 0)
    def _():
        sacc[...] = jnp.zeros_like(sacc)
        xacc[...] = jnp.full_like(xacc, NEG)

    sacc[0:1, :] += jnp.sum(jnp.where(valid, xb, 0.0), axis=0, keepdims=True)
    xacc[0:1, :] = jnp.maximum(
        xacc[0:1, :], jnp.max(jnp.where(valid, xb, NEG), axis=0, keepdims=True))

    hb = jnp.maximum(
        jnp.dot(xb, w1_ref[...], preferred_element_type=F32) + b1_ref[...], 0.0)
    pri_ref[...] = jnp.tanh(
        jnp.dot(hb[:, 0:32], w2p_ref[...], preferred_element_type=F32) + b2p_ref[...])
    coop_ref[...] = _sigmoid(
        jnp.dot(hb[:, 32:64], w2c_ref[...], preferred_element_type=F32) + b2c_ref[...])
    urg_ref[...] = _sigmoid(
        jnp.dot(hb[:, 64:96], w2u_ref[...], preferred_element_type=F32) + b2u_ref[...])
    saf_ref[...] = _sigmoid(
        jnp.dot(hb[:, 96:128], w2f_ref[...], preferred_element_type=F32) + b2f_ref[...])
    z = jnp.dot(hb[:, 128:160], w2s_ref[...], preferred_element_type=F32) + b2s_ref[...]
    zm = jnp.max(z, axis=1, keepdims=True)
    ez = jnp.exp(z - zm)
    strat_ref[...] = ez / jnp.sum(ez, axis=1, keepdims=True)

    @pl.when(i == NP // NB - 1)
    def _():
        ps = sacc[0:1, :]
        pm = ps / float(N0)
        px = xacc[0:1, :]
        gi = jnp.concatenate([pm, px, ps, u_ref[...]], axis=1)
        g0 = jnp.maximum(
            jnp.dot(gi, gpw_ref[...], preferred_element_type=F32) + gpb_ref[...], 0.0)
        mean = jnp.mean(g0, axis=1, keepdims=True)
        var = jnp.mean((g0 - mean) ** 2, axis=1, keepdims=True)
        g = (g0 - mean) * lax.rsqrt(var + 1e-5) * gplg_ref[...] + gplb_ref[...]
        gh = jnp.maximum(
            jnp.dot(g, gw1_ref[...], preferred_element_type=F32) + gb1_ref[...], 0.0)
        gs_ref[...] = jnp.tanh(
            jnp.dot(gh, gw2_ref[...], preferred_element_type=F32) + gb2_ref[...])


def _final(x, u, gp_W, gp_b, gp_ln_g, gp_ln_b, w1_all, b1_all, heads, glob):
    (w2p, b2p), (w2c, b2c), (w2u, b2u), (w2f, b2f), (w2s, b2s) = heads
    gw1, gb1, gw2, gb2 = glob
    full = lambda a, b: pl.BlockSpec((a, b), lambda i: (0, 0))
    return pl.pallas_call(
        _final_body,
        grid=(NP // NB,),
        in_specs=[
            pl.BlockSpec((NB, HD), lambda i: (i, 0)),
            full(1, GD), full(3 * HD + GD, GD), full(1, GD), full(1, GD),
            full(1, GD),
            full(HD, 160), full(1, 160),
            full(32, 1), full(1, 1), full(32, 1), full(1, 1),
            full(32, 1), full(1, 1), full(32, 1), full(1, 1),
            full(32, 5), full(1, 5),
            full(GD, 32), full(1, 32), full(32, 4), full(1, 4),
        ],
        out_specs=[
            pl.BlockSpec((NB, 1), lambda i: (i, 0)),
            pl.BlockSpec((NB, 1), lambda i: (i, 0)),
            pl.BlockSpec((NB, 1), lambda i: (i, 0)),
            pl.BlockSpec((NB, 1), lambda i: (i, 0)),
            pl.BlockSpec((NB, 5), lambda i: (i, 0)),
            pl.BlockSpec((1, 4), lambda i: (0, 0)),
        ],
        out_shape=[
            jax.ShapeDtypeStruct((NP, 1), F32),
            jax.ShapeDtypeStruct((NP, 1), F32),
            jax.ShapeDtypeStruct((NP, 1), F32),
            jax.ShapeDtypeStruct((NP, 1), F32),
            jax.ShapeDtypeStruct((NP, 5), F32),
            jax.ShapeDtypeStruct((1, 4), F32),
        ],
        scratch_shapes=[
            pltpu.VMEM((8, HD), F32),
            pltpu.VMEM((8, HD), F32),
        ],
    )(x, u.reshape(1, GD), gp_W, gp_b.reshape(1, GD),
      gp_ln_g.reshape(1, GD), gp_ln_b.reshape(1, GD), w1_all, b1_all,
      w2p, b2p.reshape(1, 1), w2c, b2c.reshape(1, 1),
      w2u, b2u.reshape(1, 1), w2f, b2f.reshape(1, 1),
      w2s, b2s.reshape(1, 5), gw1, gb1.reshape(1, 32), gw2, gb2.reshape(1, 4))


_MESH = plsc.VectorSubcoreMesh(core_axis_name="c", subcore_axis_name="s")
_CP = pltpu.CompilerParams(use_tc_tiling_on_sc=False)
if "needs_layout_passes" in pltpu.CompilerParams.__dataclass_fields__:
    _CP = dataclasses.replace(_CP, needs_layout_passes=False)
KC = 128
RPS = NP // 16
CH0 = EP // (32 * KC)
CHB = EP // (16 * KC)


@functools.partial(
    pl.kernel,
    out_type=jax.ShapeDtypeStruct((2 * NP, 16), F32),
    mesh=_MESH,
    compiler_params=_CP,
    scratch_types=[
        pltpu.VMEM((KC,), jnp.int32),
        pltpu.VMEM((KC, 16), F32),
        pltpu.VMEM_SHARED((NP, 16), F32),
    ],
)
def _p0_sc(eap_hbm, dpad_hbm, zeros_hbm, out_hbm, idx_v, row_v, table_sh):
    c = lax.axis_index("c")
    s = lax.axis_index("s")
    w = c * 16 + s
    r0 = s * RPS
    pltpu.sync_copy(zeros_hbm.at[pl.ds(r0, RPS)], table_sh.at[pl.ds(r0, RPS)])
    plsc.subcore_barrier()

    @pl.loop(0, CH0)
    def _(t):
        base = (w * CH0 + t) * KC
        pltpu.sync_copy(dpad_hbm.at[pl.ds(base, KC)], idx_v)
        pltpu.sync_copy(eap_hbm.at[pl.ds(base, KC)], row_v)
        pltpu.sync_copy(row_v, table_sh.at[idx_v], add=True)

    plsc.subcore_barrier()
    pltpu.sync_copy(table_sh.at[pl.ds(r0, RPS)],
                    out_hbm.at[pl.ds(c * NP + r0, RPS)])


def _p0_jnp(eap, dpad, zeros16):
    t0 = jax.ops.segment_sum(eap, dpad, num_segments=NP)
    return jnp.concatenate([t0, jnp.zeros((NP, 16), F32)], axis=0)


def _pa_jnp(spad, dpad, table, aeT, zeros4):
    a_src = table[spad, 0:4]
    a_dst = table[dpad, 4:8]
    m = table[dpad, 8:12]
    al = a_src + a_dst + aeT.T
    al = jnp.where(al >= 0.0, al, 0.2 * al)
    ev = jnp.exp(al - m)
    dn = jax.ops.segment_sum(ev, dpad, num_segments=NP)
    dn = jnp.pad(dn, ((0, 0), (0, 12)))
    return ev.T, jnp.concatenate([dn, jnp.zeros((NP, 16), F32)], axis=0)


def _pb_jnp(spad, dpad, xhp, ev, zeros16):
    accs = []
    for h in range(4):
        xh_h = xhp[h // 2, :, (h % 2) * 16:(h % 2) * 16 + 16]
        msg = xh_h[spad] * ev[h][:, None]
        accs.append(jax.ops.segment_sum(msg, dpad, num_segments=NP))
    return jnp.concatenate(accs, axis=0)


def _p0(eap, dpad, zeros16):
    return _p0_sc(eap, dpad, zeros16)


@functools.partial(
    pl.kernel,
    out_type=[
        jax.ShapeDtypeStruct((4, EP), F32),
        jax.ShapeDtypeStruct((2 * NP, 16), F32),
    ],
    mesh=_MESH,
    compiler_params=_CP,
    scratch_types=[
        pltpu.VMEM((KC,), jnp.int32), pltpu.VMEM((KC,), jnp.int32),
        pltpu.VMEM((KC,), jnp.int32), pltpu.VMEM((KC,), jnp.int32),
        pltpu.VMEM((KC, 16), F32), pltpu.VMEM((KC, 16), F32),
        pltpu.VMEM((KC, 16), F32), pltpu.VMEM((KC, 16), F32),
        pltpu.VMEM((4, KC), F32), pltpu.VMEM((4, KC), F32),
        pltpu.VMEM((4, KC), F32), pltpu.VMEM((4, KC), F32),
        pltpu.VMEM((KC, 16), F32),
        pltpu.VMEM_SHARED((NP, 16), F32),
        pltpu.SemaphoreType.DMA, pltpu.SemaphoreType.DMA,
        pltpu.SemaphoreType.DMA, pltpu.SemaphoreType.DMA,
        pltpu.SemaphoreType.DMA, pltpu.SemaphoreType.DMA,
        pltpu.SemaphoreType.DMA, pltpu.SemaphoreType.DMA,
        pltpu.SemaphoreType.DMA, pltpu.SemaphoreType.DMA,
        pltpu.SemaphoreType.DMA, pltpu.SemaphoreType.DMA,
    ],
)
def _pa_sc(spad_hbm, dpad_hbm, table_hbm, aeT_hbm, zeros16_hbm,
           ev_hbm, dn_hbm,
           sidx0, sidx1, didx0, didx1, srow0, srow1, drow0, drow1,
           aebuf0, aebuf1, evbuf0, evbuf1, dnbuf, dnsh,
           sS0, sS1, sD0, sD1, sA0, sA1, sGS0, sGS1, sGD0, sGD1, sE0, sE1):
    c = lax.axis_index("c")
    s = lax.axis_index("s")
    w = c * 16 + s
    r0 = s * RPS
    iota = lax.iota(jnp.int32, 16)
    sidx = (sidx0, sidx1)
    didx = (didx0, didx1)
    srow = (srow0, srow1)
    drow = (drow0, drow1)
    aebuf = (aebuf0, aebuf1)
    evbuf = (evbuf0, evbuf1)
    sS, sD, sA = (sS0, sS1), (sD0, sD1), (sA0, sA1)
    sGS, sGD, sE = (sGS0, sGS1), (sGD0, sGD1), (sE0, sE1)
    pltpu.sync_copy(zeros16_hbm.at[pl.ds(r0, RPS)], dnsh.at[pl.ds(r0, RPS)])
    pltpu.sync_copy(zeros16_hbm.at[pl.ds(0, KC)], dnbuf)
    plsc.subcore_barrier()

    def s_desc(t, b):
        base = (w * CH0 + t) * KC
        return pltpu.make_async_copy(
            spad_hbm.at[pl.ds(base, KC)], sidx[b], sS[b])

    def d_desc(t, b):
        base = (w * CH0 + t) * KC
        return pltpu.make_async_copy(
            dpad_hbm.at[pl.ds(base, KC)], didx[b], sD[b])

    def a_desc(t, b):
        base = (w * CH0 + t) * KC
        return pltpu.make_async_copy(
            aeT_hbm.at[:, pl.ds(base, KC)], aebuf[b], sA[b])

    def gs_desc(b):
        return pltpu.make_async_copy(table_hbm.at[sidx[b]], srow[b], sGS[b])

    def gd_desc(b):
        return pltpu.make_async_copy(table_hbm.at[didx[b]], drow[b], sGD[b])

    def e_desc(t, b):
        base = (w * CH0 + t) * KC
        return pltpu.make_async_copy(
            evbuf[b], ev_hbm.at[:, pl.ds(base, KC)], sE[b])

    def issue_inputs(t, b):
        s_desc(t, b).start()
        d_desc(t, b).start()
        a_desc(t, b).start()

    def start_gathers(b):
        gs_desc(b).start()
        gd_desc(b).start()

    issue_inputs(0, 0)
    issue_inputs(1, 1)
    s_desc(0, 0).wait()
    d_desc(0, 0).wait()
    start_gathers(0)

    @pl.loop(0, CH0 // 2)
    def _(tt):
        for b in range(2):
            t = tt * 2 + b

            @pl.when(t + 1 < CH0)
            def _():
                s_desc(t + 1, 1 - b).wait()
                d_desc(t + 1, 1 - b).wait()
                start_gathers(1 - b)

            gs_desc(b).wait()
            gd_desc(b).wait()
            a_desc(t, b).wait()

            @pl.when(t >= 2)
            def _():
                e_desc(t - 2, b).wait()

            for g in range(8):
                rows = g * 16 + iota
                for h in range(4):
                    hcol = jnp.full((16,), h, jnp.int32)
                    asrc = plsc.load_gather(srow[b], [rows, hcol])
                    adst = plsc.load_gather(drow[b], [rows, hcol + 4])
                    mh = plsc.load_gather(drow[b], [rows, hcol + 8])
                    ae = aebuf[b][h, pl.ds(g * 16, 16)]
                    al = asrc + adst + ae
                    al = jnp.where(al >= 0.0, al, 0.2 * al)
                    evh = jnp.exp(al - mh)
                    evbuf[b][h, pl.ds(g * 16, 16)] = evh
                    plsc.store_scatter(dnbuf, [rows, hcol], evh)
            e_desc(t, b).start()
            pltpu.sync_copy(dnbuf, dnsh.at[didx[b]], add=True)

            @pl.when(t + 2 < CH0)
            def _():
                issue_inputs(t + 2, b)

    e_desc(CH0 - 2, 0).wait()
    e_desc(CH0 - 1, 1).wait()
    plsc.subcore_barrier()
    pltpu.sync_copy(dnsh.at[pl.ds(r0, RPS)],
                    dn_hbm.at[pl.ds(c * NP + r0, RPS)])


def _pa(spad, dpad, table, aeT, zeros16):
    return _pa_sc(spad, dpad, table, aeT, zeros16)


@functools.partial(
    pl.kernel,
    out_type=jax.ShapeDtypeStruct((4 * NP, 16), F32),
    mesh=_MESH,
    compiler_params=_CP,
    scratch_types=[
        pltpu.VMEM((KC,), jnp.int32), pltpu.VMEM((KC,), jnp.int32),
        pltpu.VMEM((KC,), jnp.int32), pltpu.VMEM((KC,), jnp.int32),
        pltpu.VMEM((KC, 32), F32), pltpu.VMEM((KC, 32), F32),
        pltpu.VMEM((KC,), F32), pltpu.VMEM((KC,), F32),
        pltpu.VMEM((KC,), F32), pltpu.VMEM((KC,), F32),
        pltpu.VMEM((KC, 16), F32), pltpu.VMEM((KC, 16), F32),
        pltpu.VMEM((KC, 16), F32), pltpu.VMEM((KC, 16), F32),
        pltpu.VMEM_SHARED((NP, 16), F32),
        pltpu.VMEM_SHARED((NP, 16), F32),
        pltpu.SemaphoreType.DMA, pltpu.SemaphoreType.DMA,
        pltpu.SemaphoreType.DMA, pltpu.SemaphoreType.DMA,
        pltpu.SemaphoreType.DMA, pltpu.SemaphoreType.DMA,
        pltpu.SemaphoreType.DMA, pltpu.SemaphoreType.DMA,
        pltpu.SemaphoreType.DMA, pltpu.SemaphoreType.DMA,
        pltpu.SemaphoreType.DMA, pltpu.SemaphoreType.DMA,
        pltpu.SemaphoreType.DMA, pltpu.SemaphoreType.DMA,
    ],
)
def _pb_sc(spad2_hbm, dpad_hbm, xhpf_hbm, ev1_hbm, zeros_hbm,
           acc_hbm,
           sidx0, sidx1, didx0, didx1, xrow0, xrow1,
           ev00, ev01, ev10, ev11,
           msg00, msg01, msg10, msg11, acc0_sh, acc1_sh,
           sS0, sS1, sD0, sD1, sX0, sX1, sE00, sE01, sE10, sE11,
           sM00, sM01, sM10, sM11):
    c = lax.axis_index("c")
    s = lax.axis_index("s")
    r0 = s * RPS
    iota = lax.iota(jnp.int32, 16)
    sidx = (sidx0, sidx1)
    didx = (didx0, didx1)
    xrow = (xrow0, xrow1)
    ev0 = (ev00, ev01)
    ev1 = (ev10, ev11)
    msg0 = (msg00, msg01)
    msg1 = (msg10, msg11)
    sS, sD, sX = (sS0, sS1), (sD0, sD1), (sX0, sX1)
    sE0, sE1 = (sE00, sE01), (sE10, sE11)
    sM0, sM1 = (sM00, sM01), (sM10, sM11)
    pltpu.sync_copy(zeros_hbm.at[pl.ds(r0, RPS)], acc0_sh.at[pl.ds(r0, RPS)])
    pltpu.sync_copy(zeros_hbm.at[pl.ds(r0, RPS)], acc1_sh.at[pl.ds(r0, RPS)])
    plsc.subcore_barrier()

    def s_desc(t, b):
        base = (s * CHB + t) * KC
        return pltpu.make_async_copy(
            spad2_hbm.at[pl.ds(c * EP + base, KC)], sidx[b], sS[b])

    def d_desc(t, b):
        base = (s * CHB + t) * KC
        return pltpu.make_async_copy(
            dpad_hbm.at[pl.ds(base, KC)], didx[b], sD[b])

    def e_descs(t, b):
        base = (s * CHB + t) * KC
        return (pltpu.make_async_copy(
                    ev1_hbm.at[pl.ds((2 * c) * EP + base, KC)], ev0[b], sE0[b]),
                pltpu.make_async_copy(
                    ev1_hbm.at[pl.ds((2 * c + 1) * EP + base, KC)], ev1[b], sE1[b]))

    def x_desc(b):
        return pltpu.make_async_copy(xhpf_hbm.at[sidx[b]], xrow[b], sX[b])

    def m_descs(b):
        return (pltpu.make_async_copy(msg0[b], acc0_sh.at[didx[b]], sM0[b]),
                pltpu.make_async_copy(msg1[b], acc1_sh.at[didx[b]], sM1[b]))

    def issue_inputs(t, b):
        s_desc(t, b).start()
        ea, eb = e_descs(t, b)
        ea.start()
        eb.start()

    issue_inputs(0, 0)
    issue_inputs(1, 1)
    s_desc(0, 0).wait()
    x_desc(0).start()

    @pl.loop(0, CHB // 2)
    def _(tt):
        for b in range(2):
            t = tt * 2 + b

            @pl.when(t + 1 < CHB)
            def _():
                s_desc(t + 1, 1 - b).wait()
                x_desc(1 - b).start()

            x_desc(b).wait()
            ea, eb = e_descs(t, b)
            ea.wait()
            eb.wait()

            @pl.when(t >= 2)
            def _():
                m0p, m1p = m_descs(b)
                m0p.wait()
                m1p.wait()

            d_desc(t, b).start()
            for g in range(8):
                rows = g * 16 + iota
                ev0v = ev0[b][pl.ds(g * 16, 16)]
                ev1v = ev1[b][pl.ds(g * 16, 16)]
                for cc in range(16):
                    ccol = jnp.full((16,), cc, jnp.int32)
                    c0 = plsc.load_gather(xrow[b], [rows, ccol])
                    plsc.store_scatter(msg0[b], [rows, ccol], c0 * ev0v)
                    c1 = plsc.load_gather(xrow[b], [rows, ccol + 16])
                    plsc.store_scatter(msg1[b], [rows, ccol], c1 * ev1v)
            d_desc(t, b).wait()
            m0, m1 = m_descs(b)
            m0.start(add=True)
            m1.start(add=True)

            @pl.when(t + 2 < CHB)
            def _():
                issue_inputs(t + 2, b)

    for b in range(2):
        m0, m1 = m_descs(b)
        m0.wait()
        m1.wait()
    plsc.subcore_barrier()
    pltpu.sync_copy(acc0_sh.at[pl.ds(r0, RPS)],
                    acc_hbm.at[pl.ds((2 * c) * NP + r0, RPS)])
    pltpu.sync_copy(acc1_sh.at[pl.ds(r0, RPS)],
                    acc_hbm.at[pl.ds((2 * c + 1) * NP + r0, RPS)])


def _pb(spad2, dpad, xhp, ev, zeros16):
    return _pb_sc(spad2, dpad, xhp.reshape(2 * NP, 32), ev.reshape(4 * EP),
                  zeros16)


def kernel(x, edge_index, edge_attr, u, enc_W, enc_b,
           gat0_lin_W, gat0_att_src, gat0_att_dst, gat0_att_edge, gat0_edge_W,
           gat0_bias, gat0_ln_g, gat0_ln_b,
           gat1_lin_W, gat1_att_src, gat1_att_dst, gat1_att_edge, gat1_edge_W,
           gat1_bias, gat1_ln_g, gat1_ln_b,
           gat2_lin_W, gat2_att_src, gat2_att_dst, gat2_att_edge, gat2_edge_W,
           gat2_bias, gat2_ln_g, gat2_ln_b,
           gp_W, gp_b, gp_ln_g, gp_ln_b,
           head_priority_W1, head_priority_b1, head_priority_W2, head_priority_b2,
           head_cooperation_W1, head_cooperation_b1, head_cooperation_W2, head_cooperation_b2,
           head_urgency_W1, head_urgency_b1, head_urgency_W2, head_urgency_b2,
           head_safety_W1, head_safety_b1, head_safety_W2, head_safety_b2,
           head_strategy_W1, head_strategy_b1, head_strategy_W2, head_strategy_b2,
           glob_W1, glob_b1, glob_W2, glob_b2):
    gat = [
        (gat0_lin_W, gat0_att_src, gat0_att_dst, gat0_att_edge, gat0_edge_W,
         gat0_bias, gat0_ln_g, gat0_ln_b),
        (gat1_lin_W, gat1_att_src, gat1_att_dst, gat1_att_edge, gat1_edge_W,
         gat1_bias, gat1_ln_g, gat1_ln_b),
        (gat2_lin_W, gat2_att_src, gat2_att_dst, gat2_att_edge, gat2_edge_W,
         gat2_bias, gat2_ln_g, gat2_ln_b),
    ]

    xp = jnp.pad(x, ((0, NP - N0), (0, 0)))
    spad = jnp.concatenate(
        [edge_index[0], jnp.zeros((EP - E0,), jnp.int32)])
    dpad = jnp.concatenate(
        [edge_index[1], jnp.full((EP - E0,), TRASH, jnp.int32)])
    spad2 = jnp.concatenate([spad, spad + NP])
    eap_raw = jnp.pad(edge_attr, ((0, EP - E0), (0, 0)))
    zeros16 = jnp.zeros((NP, 16), F32)

    w2s_l = [(gw[4].reshape(ED, H, C) * gw[3][None]).sum(-1) for gw in gat]
    w2all = jnp.concatenate(w2s_l, axis=1)
    w2pad = [jnp.pad(w2, ((0, 6), (0, 0))) for w2 in w2s_l]
    sind = jnp.repeat(jnp.eye(4, dtype=F32), 16, axis=0)

    eap, aeT0, aeT1, aeT2 = _edge_prep(eap_raw, w2all)
    aeTs = [aeT0, aeT1, aeT2]
    t = _p0(eap, dpad, zeros16)
    la = _loopattr(t)

    xcur = _enc(xp, enc_W, enc_b)
    for l in range(NL):
        lin_W, att_src, att_dst, att_edge, edge_W, bias, ln_g, ln_b = gat[l]
        tab, xhp = _prep(xcur, la, lin_W,
                         att_src.reshape(1, HD), att_dst.reshape(1, HD),
                         w2pad[l], sind)
        ev, dn = _pa(spad, dpad, tab, aeTs[l], zeros16)
        acc = _pb(spad2, dpad, xhp, ev, zeros16)
        xcur = _combine(xcur, xhp, acc, dn, bias, ln_g, ln_b)

    w1_all = jnp.concatenate(
        [head_priority_W1, head_cooperation_W1, head_urgency_W1,
         head_safety_W1, head_strategy_W1], axis=1)
    b1_all = jnp.concatenate(
        [head_priority_b1, head_cooperation_b1, head_urgency_b1,
         head_safety_b1, head_strategy_b1]).reshape(1, 160)
    heads = [(head_priority_W2, head_priority_b2),
             (head_cooperation_W2, head_cooperation_b2),
             (head_urgency_W2, head_urgency_b2),
             (head_safety_W2, head_safety_b2),
             (head_strategy_W2, head_strategy_b2)]
    glob = (glob_W1, glob_b1, glob_W2, glob_b2)
    pri, coop, urg, saf, strat, gs = _final(
        xcur, u, gp_W, gp_b, gp_ln_g, gp_ln_b, w1_all, b1_all, heads, glob)
    return (pri[:N0], coop[:N0], urg[:N0], saf[:N0], strat[:N0],
            gs.reshape(GD // 2))


# speedup vs baseline: 37.8331x; 1.0000x over previous
import dataclasses
import functools

import jax
import jax.numpy as jnp
from jax import lax
from jax.experimental import pallas as pl
from jax.experimental.pallas import tpu as pltpu
from jax.experimental.pallas import tpu_sc as plsc

N0 = 50000
E0 = 800000
ND, ED, GD, HD = 15, 10, 8, 64
H, C, NL = 4, 16, 3
NB = 1024
EB = 4096
NP = 49 * NB
EP = 196 * EB
TRASH = NP - 1
NEG = -3.4e38
F32 = jnp.float32


def _enc_body(x_ref, w_ref, b_ref, o_ref):
    i = pl.program_id(0)
    y = jnp.dot(x_ref[...], w_ref[...], preferred_element_type=F32) + b_ref[...]
    y = jnp.maximum(y, 0.0)
    rid = i * NB + lax.broadcasted_iota(jnp.int32, (NB, 1), 0)
    o_ref[...] = jnp.where(rid < N0, y, 0.0)


def _enc(xp, enc_W, enc_b):
    return pl.pallas_call(
        _enc_body,
        grid=(NP // NB,),
        in_specs=[
            pl.BlockSpec((NB, ND), lambda i: (i, 0)),
            pl.BlockSpec((ND, HD), lambda i: (0, 0)),
            pl.BlockSpec((1, HD), lambda i: (0, 0)),
        ],
        out_specs=pl.BlockSpec((NB, HD), lambda i: (i, 0)),
        out_shape=jax.ShapeDtypeStruct((NP, HD), F32),
    )(xp, enc_W, enc_b.reshape(1, HD))


def _edge_body(ea_ref, w2_ref, eap_ref, a0_ref, a1_ref, a2_ref):
    i = pl.program_id(0)
    ea = ea_ref[...]
    tT = lax.dot_general(w2_ref[...], ea, (((0,), (1,)), ((), ())),
                         preferred_element_type=F32)
    rid = i * EB + lax.broadcasted_iota(jnp.int32, (EB, 1), 0)
    one = jnp.where(rid < E0, 1.0, 0.0)
    eap_ref[...] = jnp.concatenate(
        [ea, one, jnp.zeros((EB, 5), F32)], axis=1)
    a0_ref[...] = tT[0:4]
    a1_ref[...] = tT[4:8]
    a2_ref[...] = tT[8:12]


def _edge_prep(eap_raw, w2all):
    aspec = pl.BlockSpec((4, EB), lambda i: (0, i))
    return pl.pallas_call(
        _edge_body,
        grid=(EP // EB,),
        in_specs=[
            pl.BlockSpec((EB, ED), lambda i: (i, 0)),
            pl.BlockSpec((ED, 12), lambda i: (0, 0)),
        ],
        out_specs=[pl.BlockSpec((EB, 16), lambda i: (i, 0)), aspec, aspec, aspec],
        out_shape=[
            jax.ShapeDtypeStruct((EP, 16), F32),
            jax.ShapeDtypeStruct((4, EP), F32),
            jax.ShapeDtypeStruct((4, EP), F32),
            jax.ShapeDtypeStruct((4, EP), F32),
        ],
    )(eap_raw, w2all)


def _loopattr_body(t0_ref, t1_ref, o_ref):
    s = t0_ref[...] + t1_ref[...]
    deg = jnp.maximum(s[:, 10:11], 1.0)
    o_ref[...] = s / deg


def _loopattr(t):
    return pl.pallas_call(
        _loopattr_body,
        grid=(NP // NB,),
        in_specs=[
            pl.BlockSpec((NB, 16), lambda i: (i, 0)),
            pl.BlockSpec((NB, 16), lambda i: (i + NP // NB, 0)),
        ],
        out_specs=pl.BlockSpec((NB, 16), lambda i: (i, 0)),
        out_shape=jax.ShapeDtypeStruct((NP, 16), F32),
    )(t, t)


def _prep_body(x_ref, la_ref, w_ref, asf_ref, adf_ref, w2p_ref, s_ref,
               tab_ref, xhp_ref):
    xh = jnp.dot(x_ref[...], w_ref[...], preferred_element_type=F32)
    a_src = jnp.dot(xh * asf_ref[...], s_ref[...], preferred_element_type=F32)
    a_dst = jnp.dot(xh * adf_ref[...], s_ref[...], preferred_element_type=F32)
    ael = jnp.dot(la_ref[...], w2p_ref[...], preferred_element_type=F32)
    am = a_src + a_dst + ael
    m = jnp.where(am >= 0.0, am, 0.2 * am)
    tab_ref[...] = jnp.concatenate(
        [a_src, a_dst, m, jnp.zeros((NB, 4), F32)], axis=1)
    xhp_ref[0, :, :] = xh[:, 0:32]
    xhp_ref[1, :, :] = xh[:, 32:64]


def _prep(x, la, lin_W, asf, adf, w2pad, sind):
    return pl.pallas_call(
        _prep_body,
        grid=(NP // NB,),
        in_specs=[
            pl.BlockSpec((NB, HD), lambda i: (i, 0)),
            pl.BlockSpec((NB, 16), lambda i: (i, 0)),
            pl.BlockSpec((HD, HD), lambda i: (0, 0)),
            pl.BlockSpec((1, HD), lambda i: (0, 0)),
            pl.BlockSpec((1, HD), lambda i: (0, 0)),
            pl.BlockSpec((16, 4), lambda i: (0, 0)),
            pl.BlockSpec((HD, 4), lambda i: (0, 0)),
        ],
        out_specs=[
            pl.BlockSpec((NB, 16), lambda i: (i, 0)),
            pl.BlockSpec((2, NB, 32), lambda i: (0, i, 0)),
        ],
        out_shape=[
            jax.ShapeDtypeStruct((NP, 16), F32),
            jax.ShapeDtypeStruct((2, NP, 32), F32),
        ],
    )(x, la, lin_W, asf, adf, w2pad, sind)


def _combine_body(x_ref, xhp_ref, a0, a1, a2, a3, d0, d1, b_ref, g_ref,
                  bb_ref, o_ref):
    denom = d0[...][:, 0:4] + d1[...][:, 0:4] + 1.0
    accs = (a0, a1, a2, a3)
    msgs = []
    for h in range(4):
        xh_h = xhp_ref[h // 2, :, (h % 2) * 16:(h % 2) * 16 + 16]
        msgs.append((accs[h][...] + xh_h) / denom[:, h:h + 1])
    y = x_ref[...] + jnp.concatenate(msgs, axis=1) + b_ref[...]
    mean = jnp.mean(y, axis=1, keepdims=True)
    var = jnp.mean((y - mean) ** 2, axis=1, keepdims=True)
    o_ref[...] = (y - mean) * lax.rsqrt(var + 1e-5) * g_ref[...] + bb_ref[...]


def _combine(x, xhp, acc, dn, bias, ln_g, ln_b):
    nblk = NP // NB
    aspec = lambda h: pl.BlockSpec((NB, 16), lambda i, h=h: (i + h * nblk, 0))
    dspec = lambda c: pl.BlockSpec((NB, 16), lambda i, c=c: (i + c * nblk, 0))
    return pl.pallas_call(
        _combine_body,
        grid=(nblk,),
        in_specs=[
            pl.BlockSpec((NB, HD), lambda i: (i, 0)),
            pl.BlockSpec((2, NB, 32), lambda i: (0, i, 0)),
            aspec(0), aspec(1), aspec(2), aspec(3),
            dspec(0), dspec(1),
            pl.BlockSpec((1, HD), lambda i: (0, 0)),
            pl.BlockSpec((1, HD), lambda i: (0, 0)),
            pl.BlockSpec((1, HD), lambda i: (0, 0)),
        ],
        out_specs=pl.BlockSpec((NB, HD), lambda i: (i, 0)),
        out_shape=jax.ShapeDtypeStruct((NP, HD), F32),
    )(x, xhp, acc, acc, acc, acc, dn, dn,
      bias.reshape(1, HD), ln_g.reshape(1, HD), ln_b.reshape(1, HD))


def _sigmoid(z):
    return 1.0 / (1.0 + jnp.exp(-z))


def _final_body(x_ref, u_ref, gpw_ref, gpb_ref, gplg_ref, gplb_ref,
                w1_ref, b1_ref, w2p_ref, b2p_ref, w2c_ref, b2c_ref,
                w2u_ref, b2u_ref, w2f_ref, b2f_ref, w2s_ref, b2s_ref,
                gw1_ref, gb1_ref, gw2_ref, gb2_ref,
                pri_ref, coop_ref, urg_ref, saf_ref, strat_ref, gs_ref,
                sacc, xacc):
    i = pl.program_id(0)
    xb = x_ref[...]
    rid = i * NB + lax.broadcasted_iota(jnp.int32, (NB, 1), 0)
    valid = rid < N0

    @pl.when(i == 0)
    def _():
        sacc[...] = jnp.zeros_like(sacc)
        xacc[...] = jnp.full_like(xacc, NEG)

    sacc[0:1, :] += jnp.sum(jnp.where(valid, xb, 0.0), axis=0, keepdims=True)
    xacc[0:1, :] = jnp.maximum(
        xacc[0:1, :], jnp.max(jnp.where(valid, xb, NEG), axis=0, keepdims=True))

    hb = jnp.maximum(
        jnp.dot(xb, w1_ref[...], preferred_element_type=F32) + b1_ref[...], 0.0)
    pri_ref[...] = jnp.tanh(
        jnp.dot(hb[:, 0:32], w2p_ref[...], preferred_element_type=F32) + b2p_ref[...])
    coop_ref[...] = _sigmoid(
        jnp.dot(hb[:, 32:64], w2c_ref[...], preferred_element_type=F32) + b2c_ref[...])
    urg_ref[...] = _sigmoid(
        jnp.dot(hb[:, 64:96], w2u_ref[...], preferred_element_type=F32) + b2u_ref[...])
    saf_ref[...] = _sigmoid(
        jnp.dot(hb[:, 96:128], w2f_ref[...], preferred_element_type=F32) + b2f_ref[...])
    z = jnp.dot(hb[:, 128:160], w2s_ref[...], preferred_element_type=F32) + b2s_ref[...]
    zm = jnp.max(z, axis=1, keepdims=True)
    ez = jnp.exp(z - zm)
    strat_ref[...] = ez / jnp.sum(ez, axis=1, keepdims=True)

    @pl.when(i == NP // NB - 1)
    def _():
        ps = sacc[0:1, :]
        pm = ps / float(N0)
        px = xacc[0:1, :]
        gi = jnp.concatenate([pm, px, ps, u_ref[...]], axis=1)
        g0 = jnp.maximum(
            jnp.dot(gi, gpw_ref[...], preferred_element_type=F32) + gpb_ref[...], 0.0)
        mean = jnp.mean(g0, axis=1, keepdims=True)
        var = jnp.mean((g0 - mean) ** 2, axis=1, keepdims=True)
        g = (g0 - mean) * lax.rsqrt(var + 1e-5) * gplg_ref[...] + gplb_ref[...]
        gh = jnp.maximum(
            jnp.dot(g, gw1_ref[...], preferred_element_type=F32) + gb1_ref[...], 0.0)
        gs_ref[...] = jnp.tanh(
            jnp.dot(gh, gw2_ref[...], preferred_element_type=F32) + gb2_ref[...])


def _final(x, u, gp_W, gp_b, gp_ln_g, gp_ln_b, w1_all, b1_all, heads, glob):
    (w2p, b2p), (w2c, b2c), (w2u, b2u), (w2f, b2f), (w2s, b2s) = heads
    gw1, gb1, gw2, gb2 = glob
    full = lambda a, b: pl.BlockSpec((a, b), lambda i: (0, 0))
    return pl.pallas_call(
        _final_body,
        grid=(NP // NB,),
        in_specs=[
            pl.BlockSpec((NB, HD), lambda i: (i, 0)),
            full(1, GD), full(3 * HD + GD, GD), full(1, GD), full(1, GD),
            full(1, GD),
            full(HD, 160), full(1, 160),
            full(32, 1), full(1, 1), full(32, 1), full(1, 1),
            full(32, 1), full(1, 1), full(32, 1), full(1, 1),
            full(32, 5), full(1, 5),
            full(GD, 32), full(1, 32), full(32, 4), full(1, 4),
        ],
        out_specs=[
            pl.BlockSpec((NB, 1), lambda i: (i, 0)),
            pl.BlockSpec((NB, 1), lambda i: (i, 0)),
            pl.BlockSpec((NB, 1), lambda i: (i, 0)),
            pl.BlockSpec((NB, 1), lambda i: (i, 0)),
            pl.BlockSpec((NB, 5), lambda i: (i, 0)),
            pl.BlockSpec((1, 4), lambda i: (0, 0)),
        ],
        out_shape=[
            jax.ShapeDtypeStruct((NP, 1), F32),
            jax.ShapeDtypeStruct((NP, 1), F32),
            jax.ShapeDtypeStruct((NP, 1), F32),
            jax.ShapeDtypeStruct((NP, 1), F32),
            jax.ShapeDtypeStruct((NP, 5), F32),
            jax.ShapeDtypeStruct((1, 4), F32),
        ],
        scratch_shapes=[
            pltpu.VMEM((8, HD), F32),
            pltpu.VMEM((8, HD), F32),
        ],
    )(x, u.reshape(1, GD), gp_W, gp_b.reshape(1, GD),
      gp_ln_g.reshape(1, GD), gp_ln_b.reshape(1, GD), w1_all, b1_all,
      w2p, b2p.reshape(1, 1), w2c, b2c.reshape(1, 1),
      w2u, b2u.reshape(1, 1), w2f, b2f.reshape(1, 1),
      w2s, b2s.reshape(1, 5), gw1, gb1.reshape(1, 32), gw2, gb2.reshape(1, 4))


_MESH = plsc.VectorSubcoreMesh(core_axis_name="c", subcore_axis_name="s")
_CP = pltpu.CompilerParams(use_tc_tiling_on_sc=False)
if "needs_layout_passes" in pltpu.CompilerParams.__dataclass_fields__:
    _CP = dataclasses.replace(_CP, needs_layout_passes=False)
KC = 128
RPS = NP // 16
CH0 = EP // (32 * KC)
CHB = EP // (16 * KC)


@functools.partial(
    pl.kernel,
    out_type=jax.ShapeDtypeStruct((2 * NP, 16), F32),
    mesh=_MESH,
    compiler_params=_CP,
    scratch_types=[
        pltpu.VMEM((KC,), jnp.int32),
        pltpu.VMEM((KC, 16), F32),
        pltpu.VMEM_SHARED((NP, 16), F32),
    ],
)
def _p0_sc(eap_hbm, dpad_hbm, zeros_hbm, out_hbm, idx_v, row_v, table_sh):
    c = lax.axis_index("c")
    s = lax.axis_index("s")
    w = c * 16 + s
    r0 = s * RPS
    pltpu.sync_copy(zeros_hbm.at[pl.ds(r0, RPS)], table_sh.at[pl.ds(r0, RPS)])
    plsc.subcore_barrier()

    @pl.loop(0, CH0)
    def _(t):
        base = (w * CH0 + t) * KC
        pltpu.sync_copy(dpad_hbm.at[pl.ds(base, KC)], idx_v)
        pltpu.sync_copy(eap_hbm.at[pl.ds(base, KC)], row_v)
        pltpu.sync_copy(row_v, table_sh.at[idx_v], add=True)

    plsc.subcore_barrier()
    pltpu.sync_copy(table_sh.at[pl.ds(r0, RPS)],
                    out_hbm.at[pl.ds(c * NP + r0, RPS)])


def _p0(eap, dpad, zeros16):
    return _p0_sc(eap, dpad, zeros16)


@functools.partial(
    pl.kernel,
    out_type=[
        jax.ShapeDtypeStruct((4, EP), F32),
        jax.ShapeDtypeStruct((2 * NP, 16), F32),
    ],
    mesh=_MESH,
    compiler_params=_CP,
    scratch_types=[
        pltpu.VMEM((KC,), jnp.int32), pltpu.VMEM((KC,), jnp.int32),
        pltpu.VMEM((KC,), jnp.int32), pltpu.VMEM((KC,), jnp.int32),
        pltpu.VMEM((KC, 16), F32), pltpu.VMEM((KC, 16), F32),
        pltpu.VMEM((KC, 16), F32), pltpu.VMEM((KC, 16), F32),
        pltpu.VMEM((4, KC), F32), pltpu.VMEM((4, KC), F32),
        pltpu.VMEM((4, KC), F32), pltpu.VMEM((4, KC), F32),
        pltpu.VMEM((KC, 16), F32),
        pltpu.VMEM_SHARED((NP, 16), F32),
        pltpu.SemaphoreType.DMA, pltpu.SemaphoreType.DMA,
        pltpu.SemaphoreType.DMA, pltpu.SemaphoreType.DMA,
        pltpu.SemaphoreType.DMA, pltpu.SemaphoreType.DMA,
        pltpu.SemaphoreType.DMA, pltpu.SemaphoreType.DMA,
        pltpu.SemaphoreType.DMA, pltpu.SemaphoreType.DMA,
        pltpu.SemaphoreType.DMA, pltpu.SemaphoreType.DMA,
    ],
)
def _pa_sc(spad_hbm, dpad_hbm, table_hbm, aeT_hbm, zeros16_hbm,
           ev_hbm, dn_hbm,
           sidx0, sidx1, didx0, didx1, srow0, srow1, drow0, drow1,
           aebuf0, aebuf1, evbuf0, evbuf1, dnbuf, dnsh,
           sS0, sS1, sD0, sD1, sA0, sA1, sGS0, sGS1, sGD0, sGD1, sE0, sE1):
    c = lax.axis_index("c")
    s = lax.axis_index("s")
    w = c * 16 + s
    r0 = s * RPS
    iota = lax.iota(jnp.int32, 16)
    sidx = (sidx0, sidx1)
    didx = (didx0, didx1)
    srow = (srow0, srow1)
    drow = (drow0, drow1)
    aebuf = (aebuf0, aebuf1)
    evbuf = (evbuf0, evbuf1)
    sS, sD, sA = (sS0, sS1), (sD0, sD1), (sA0, sA1)
    sGS, sGD, sE = (sGS0, sGS1), (sGD0, sGD1), (sE0, sE1)
    pltpu.sync_copy(zeros16_hbm.at[pl.ds(r0, RPS)], dnsh.at[pl.ds(r0, RPS)])
    pltpu.sync_copy(zeros16_hbm.at[pl.ds(0, KC)], dnbuf)
    plsc.subcore_barrier()

    def s_desc(t, b):
        base = (w * CH0 + t) * KC
        return pltpu.make_async_copy(
            spad_hbm.at[pl.ds(base, KC)], sidx[b], sS[b])

    def d_desc(t, b):
        base = (w * CH0 + t) * KC
        return pltpu.make_async_copy(
            dpad_hbm.at[pl.ds(base, KC)], didx[b], sD[b])

    def a_desc(t, b):
        base = (w * CH0 + t) * KC
        return pltpu.make_async_copy(
            aeT_hbm.at[:, pl.ds(base, KC)], aebuf[b], sA[b])

    def gs_desc(b):
        return pltpu.make_async_copy(table_hbm.at[sidx[b]], srow[b], sGS[b])

    def gd_desc(b):
        return pltpu.make_async_copy(table_hbm.at[didx[b]], drow[b], sGD[b])

    def e_desc(t, b):
        base = (w * CH0 + t) * KC
        return pltpu.make_async_copy(
            evbuf[b], ev_hbm.at[:, pl.ds(base, KC)], sE[b])

    def issue_inputs(t, b):
        s_desc(t, b).start()
        d_desc(t, b).start()
        a_desc(t, b).start()

    def start_gathers(b):
        gs_desc(b).start()
        gd_desc(b).start()

    issue_inputs(0, 0)
    issue_inputs(1, 1)
    s_desc(0, 0).wait()
    d_desc(0, 0).wait()
    start_gathers(0)

    @pl.loop(0, CH0 // 2)
    def _(tt):
        for b in range(2):
            t = tt * 2 + b

            @pl.when(t + 1 < CH0)
            def _():
                s_desc(t + 1, 1 - b).wait()
                d_desc(t + 1, 1 - b).wait()
                start_gathers(1 - b)

            gs_desc(b).wait()
            gd_desc(b).wait()
            a_desc(t, b).wait()

            @pl.when(t >= 2)
            def _():
                e_desc(t - 2, b).wait()

            for g in range(8):
                rows = g * 16 + iota
                for h in range(4):
                    hcol = jnp.full((16,), h, jnp.int32)
                    asrc = plsc.load_gather(srow[b], [rows, hcol])
                    adst = plsc.load_gather(drow[b], [rows, hcol + 4])
                    mh = plsc.load_gather(drow[b], [rows, hcol + 8])
                    ae = aebuf[b][h, pl.ds(g * 16, 16)]
                    al = asrc + adst + ae
                    al = jnp.where(al >= 0.0, al, 0.2 * al)
                    evh = jnp.exp(al - mh)
                    evbuf[b][h, pl.ds(g * 16, 16)] = evh
                    plsc.store_scatter(dnbuf, [rows, hcol], evh)
            e_desc(t, b).start()
            pltpu.sync_copy(dnbuf, dnsh.at[didx[b]], add=True)

            @pl.when(t + 2 < CH0)
            def _():
                issue_inputs(t + 2, b)

    e_desc(CH0 - 2, 0).wait()
    e_desc(CH0 - 1, 1).wait()
    plsc.subcore_barrier()
    pltpu.sync_copy(dnsh.at[pl.ds(r0, RPS)],
                    dn_hbm.at[pl.ds(c * NP + r0, RPS)])


def _pa(spad, dpad, table, aeT, zeros16):
    return _pa_sc(spad, dpad, table, aeT, zeros16)


@functools.partial(
    pl.kernel,
    out_type=jax.ShapeDtypeStruct((4 * NP, 16), F32),
    mesh=_MESH,
    compiler_params=_CP,
    scratch_types=[
        pltpu.VMEM((KC,), jnp.int32), pltpu.VMEM((KC,), jnp.int32),
        pltpu.VMEM((KC,), jnp.int32), pltpu.VMEM((KC,), jnp.int32),
        pltpu.VMEM((KC, 32), F32), pltpu.VMEM((KC, 32), F32),
        pltpu.VMEM((KC,), F32), pltpu.VMEM((KC,), F32),
        pltpu.VMEM((KC,), F32), pltpu.VMEM((KC,), F32),
        pltpu.VMEM((KC, 16), F32), pltpu.VMEM((KC, 16), F32),
        pltpu.VMEM((KC, 16), F32), pltpu.VMEM((KC, 16), F32),
        pltpu.VMEM_SHARED((NP, 16), F32),
        pltpu.VMEM_SHARED((NP, 16), F32),
        pltpu.SemaphoreType.DMA, pltpu.SemaphoreType.DMA,
        pltpu.SemaphoreType.DMA, pltpu.SemaphoreType.DMA,
        pltpu.SemaphoreType.DMA, pltpu.SemaphoreType.DMA,
        pltpu.SemaphoreType.DMA, pltpu.SemaphoreType.DMA,
        pltpu.SemaphoreType.DMA, pltpu.SemaphoreType.DMA,
        pltpu.SemaphoreType.DMA, pltpu.SemaphoreType.DMA,
        pltpu.SemaphoreType.DMA, pltpu.SemaphoreType.DMA,
    ],
)
def _pb_sc(spad2_hbm, dpad_hbm, xhpf_hbm, ev1_hbm, zeros_hbm,
           acc_hbm,
           sidx0, sidx1, didx0, didx1, xrow0, xrow1,
           ev00, ev01, ev10, ev11,
           msg00, msg01, msg10, msg11, acc0_sh, acc1_sh,
           sS0, sS1, sD0, sD1, sX0, sX1, sE00, sE01, sE10, sE11,
           sM00, sM01, sM10, sM11):
    c = lax.axis_index("c")
    s = lax.axis_index("s")
    r0 = s * RPS
    iota = lax.iota(jnp.int32, 16)
    sidx = (sidx0, sidx1)
    didx = (didx0, didx1)
    xrow = (xrow0, xrow1)
    ev0 = (ev00, ev01)
    ev1 = (ev10, ev11)
    msg0 = (msg00, msg01)
    msg1 = (msg10, msg11)
    sS, sD, sX = (sS0, sS1), (sD0, sD1), (sX0, sX1)
    sE0, sE1 = (sE00, sE01), (sE10, sE11)
    sM0, sM1 = (sM00, sM01), (sM10, sM11)
    pltpu.sync_copy(zeros_hbm.at[pl.ds(r0, RPS)], acc0_sh.at[pl.ds(r0, RPS)])
    pltpu.sync_copy(zeros_hbm.at[pl.ds(r0, RPS)], acc1_sh.at[pl.ds(r0, RPS)])
    plsc.subcore_barrier()

    def s_desc(t, b):
        base = (s * CHB + t) * KC
        return pltpu.make_async_copy(
            spad2_hbm.at[pl.ds(c * EP + base, KC)], sidx[b], sS[b])

    def d_desc(t, b):
        base = (s * CHB + t) * KC
        return pltpu.make_async_copy(
            dpad_hbm.at[pl.ds(base, KC)], didx[b], sD[b])

    def e_descs(t, b):
        base = (s * CHB + t) * KC
        return (pltpu.make_async_copy(
                    ev1_hbm.at[pl.ds((2 * c) * EP + base, KC)], ev0[b], sE0[b]),
                pltpu.make_async_copy(
                    ev1_hbm.at[pl.ds((2 * c + 1) * EP + base, KC)], ev1[b], sE1[b]))

    def x_desc(b):
        return pltpu.make_async_copy(xhpf_hbm.at[sidx[b]], xrow[b], sX[b])

    def m_descs(b):
        return (pltpu.make_async_copy(msg0[b], acc0_sh.at[didx[b]], sM0[b]),
                pltpu.make_async_copy(msg1[b], acc1_sh.at[didx[b]], sM1[b]))

    def issue_inputs(t, b):
        s_desc(t, b).start()
        ea, eb = e_descs(t, b)
        ea.start()
        eb.start()

    issue_inputs(0, 0)
    issue_inputs(1, 1)
    s_desc(0, 0).wait()
    x_desc(0).start()

    @pl.loop(0, CHB // 2)
    def _(tt):
        for b in range(2):
            t = tt * 2 + b

            @pl.when(t + 1 < CHB)
            def _():
                s_desc(t + 1, 1 - b).wait()
                x_desc(1 - b).start()

            x_desc(b).wait()
            ea, eb = e_descs(t, b)
            ea.wait()
            eb.wait()

            @pl.when(t >= 2)
            def _():
                m0p, m1p = m_descs(b)
                m0p.wait()
                m1p.wait()

            d_desc(t, b).start()
            for g in range(8):
                rows = g * 16 + iota
                ev0v = ev0[b][pl.ds(g * 16, 16)]
                ev1v = ev1[b][pl.ds(g * 16, 16)]
                for cc in range(16):
                    ccol = jnp.full((16,), cc, jnp.int32)
                    c0 = plsc.load_gather(xrow[b], [rows, ccol])
                    plsc.store_scatter(msg0[b], [rows, ccol], c0 * ev0v)
                    c1 = plsc.load_gather(xrow[b], [rows, ccol + 16])
                    plsc.store_scatter(msg1[b], [rows, ccol], c1 * ev1v)
            d_desc(t, b).wait()
            m0, m1 = m_descs(b)
            m0.start(add=True)
            m1.start(add=True)

            @pl.when(t + 2 < CHB)
            def _():
                issue_inputs(t + 2, b)

    for b in range(2):
        m0, m1 = m_descs(b)
        m0.wait()
        m1.wait()
    plsc.subcore_barrier()
    pltpu.sync_copy(acc0_sh.at[pl.ds(r0, RPS)],
                    acc_hbm.at[pl.ds((2 * c) * NP + r0, RPS)])
    pltpu.sync_copy(acc1_sh.at[pl.ds(r0, RPS)],
                    acc_hbm.at[pl.ds((2 * c + 1) * NP + r0, RPS)])


def _pb(spad2, dpad, xhp, ev, zeros16):
    return _pb_sc(spad2, dpad, xhp.reshape(2 * NP, 32), ev.reshape(4 * EP),
                  zeros16)


def kernel(x, edge_index, edge_attr, u, enc_W, enc_b,
           gat0_lin_W, gat0_att_src, gat0_att_dst, gat0_att_edge, gat0_edge_W,
           gat0_bias, gat0_ln_g, gat0_ln_b,
           gat1_lin_W, gat1_att_src, gat1_att_dst, gat1_att_edge, gat1_edge_W,
           gat1_bias, gat1_ln_g, gat1_ln_b,
           gat2_lin_W, gat2_att_src, gat2_att_dst, gat2_att_edge, gat2_edge_W,
           gat2_bias, gat2_ln_g, gat2_ln_b,
           gp_W, gp_b, gp_ln_g, gp_ln_b,
           head_priority_W1, head_priority_b1, head_priority_W2, head_priority_b2,
           head_cooperation_W1, head_cooperation_b1, head_cooperation_W2, head_cooperation_b2,
           head_urgency_W1, head_urgency_b1, head_urgency_W2, head_urgency_b2,
           head_safety_W1, head_safety_b1, head_safety_W2, head_safety_b2,
           head_strategy_W1, head_strategy_b1, head_strategy_W2, head_strategy_b2,
           glob_W1, glob_b1, glob_W2, glob_b2):
    gat = [
        (gat0_lin_W, gat0_att_src, gat0_att_dst, gat0_att_edge, gat0_edge_W,
         gat0_bias, gat0_ln_g, gat0_ln_b),
        (gat1_lin_W, gat1_att_src, gat1_att_dst, gat1_att_edge, gat1_edge_W,
         gat1_bias, gat1_ln_g, gat1_ln_b),
        (gat2_lin_W, gat2_att_src, gat2_att_dst, gat2_att_edge, gat2_edge_W,
         gat2_bias, gat2_ln_g, gat2_ln_b),
    ]

    xp = jnp.pad(x, ((0, NP - N0), (0, 0)))
    spad = jnp.concatenate(
        [edge_index[0], jnp.zeros((EP - E0,), jnp.int32)])
    dpad = jnp.concatenate(
        [edge_index[1], jnp.full((EP - E0,), TRASH, jnp.int32)])
    spad2 = jnp.concatenate([spad, spad + NP])
    eap_raw = jnp.pad(edge_attr, ((0, EP - E0), (0, 0)))
    zeros16 = jnp.zeros((NP, 16), F32)

    w2s_l = [(gw[4].reshape(ED, H, C) * gw[3][None]).sum(-1) for gw in gat]
    w2all = jnp.concatenate(w2s_l, axis=1)
    w2pad = [jnp.pad(w2, ((0, 6), (0, 0))) for w2 in w2s_l]
    sind = jnp.repeat(jnp.eye(4, dtype=F32), 16, axis=0)

    eap, aeT0, aeT1, aeT2 = _edge_prep(eap_raw, w2all)
    aeTs = [aeT0, aeT1, aeT2]
    t = _p0(eap, dpad, zeros16)
    la = _loopattr(t)

    xcur = _enc(xp, enc_W, enc_b)
    for l in range(NL):
        lin_W, att_src, att_dst, att_edge, edge_W, bias, ln_g, ln_b = gat[l]
        tab, xhp = _prep(xcur, la, lin_W,
                         att_src.reshape(1, HD), att_dst.reshape(1, HD),
                         w2pad[l], sind)
        ev, dn = _pa(spad, dpad, tab, aeTs[l], zeros16)
        acc = _pb(spad2, dpad, xhp, ev, zeros16)
        xcur = _combine(xcur, xhp, acc, dn, bias, ln_g, ln_b)

    w1_all = jnp.concatenate(
        [head_priority_W1, head_cooperation_W1, head_urgency_W1,
         head_safety_W1, head_strategy_W1], axis=1)
    b1_all = jnp.concatenate(
        [head_priority_b1, head_cooperation_b1, head_urgency_b1,
         head_safety_b1, head_strategy_b1]).reshape(1, 160)
    heads = [(head_priority_W2, head_priority_b2),
             (head_cooperation_W2, head_cooperation_b2),
             (head_urgency_W2, head_urgency_b2),
             (head_safety_W2, head_safety_b2),
             (head_strategy_W2, head_strategy_b2)]
    glob = (glob_W1, glob_b1, glob_W2, glob_b2)
    pri, coop, urg, saf, strat, gs = _final(
        xcur, u, gp_W, gp_b, gp_ln_g, gp_ln_b, w1_all, b1_all, heads, glob)
    return (pri[:N0], coop[:N0], urg[:N0], saf[:N0], strat[:N0],
            gs.reshape(GD // 2))
